# Initial kernel scaffold; baseline (speedup 1.0000x reference)
import jax
import jax.numpy as jnp
from jax.experimental import pallas as pl


def kernel(t, f0, index, v0, epsilon_v, epsilon_r):
    raise NotImplementedError("write your pallas kernel here")


# speedup vs baseline: 1.1012x; 1.1012x over previous
import functools

import jax
import jax.numpy as jnp
from jax import lax
from jax.experimental import pallas as pl
from jax.experimental.pallas import tpu as pltpu
from jax.experimental.pallas import tpu_sc as plsc

N = 1600000
S = 32768
EPS = 1e-05
T_SCALE = 2.0

NC = 2
NS = 16
NW = NC * NS
RW = N // NW
RB = 2000
NB = RW // RB
MB = RB // 16
SWL = S // NW
ZL = 3 * S // NS
CL = S // NS

_params = pltpu.CompilerParams(needs_layout_passes=False)

_mesh = functools.partial(
    plsc.VectorSubcoreMesh, core_axis_name="c", subcore_axis_name="s",
    num_cores=NC, num_subcores=NS)


def _wid():
    c = lax.axis_index("c")
    s = lax.axis_index("s")
    return s * NC + c, c, s


def _patterns():
    i = lax.iota(jnp.int32, 16)
    pats = []
    for v in range(3):
        flat = i + 16 * v
        pats.append((flat // 3, flat % 3))
    return pats


def _zero_fill(ref, n):
    z = jnp.zeros((16,), jnp.float32)

    def body(k, _):
        ref[pl.ds(16 * k, 16)] = z
        return 0

    lax.fori_loop(0, n // 16, body, 0)


def _build_idx3(idx_v, i3_v, pats):
    def body(k, _):
        for v, (rp, cp) in enumerate(pats):
            seg = plsc.load_gather(idx_v, [rp + 16 * k])
            i3_v[pl.ds(48 * k + 16 * v, 16)] = seg * 3 + cp
        return 0

    lax.fori_loop(0, MB, body, 0)


def _sqrt16(x):
    y = plsc.bitcast(
        jnp.int32(0x5F3759DF) - (plsc.bitcast(x, jnp.int32) >> 1), jnp.float32)
    half = x * 0.5
    for _ in range(3):
        y = y * (1.5 - half * y * y)
    return x * y


def _floor16(x):
    t = lax.convert_element_type(
        lax.convert_element_type(x, jnp.int32), jnp.float32)
    return jnp.where(t > x, t - 1.0, t)


def _wrap_signed16(x):
    y = x + 0.5
    return (y - _floor16(y)) - 0.5


def _wrap_frac16(x):
    return x - _floor16(x)


def _sums_body(index, ev, er, psum_v, psum_r, pcnt,
               idx_v, i3_v, ev_v, er_v, ones_v, zb_v, acc_v, acc_r, acc_c):
    wid, c, s = _wid()
    pats = _patterns()

    one = jnp.full((16,), 1.0, jnp.float32)

    def ones_body(k, _):
        ones_v[pl.ds(16 * k, 16)] = one
        return 0

    lax.fori_loop(0, RB // 16, ones_body, 0)

    _zero_fill(zb_v, ZL)
    pltpu.sync_copy(zb_v, acc_v.at[pl.ds(s * ZL, ZL)])
    pltpu.sync_copy(zb_v, acc_r.at[pl.ds(s * ZL, ZL)])
    pltpu.sync_copy(zb_v.at[pl.ds(0, CL)], acc_c.at[pl.ds(s * CL, CL)])
    plsc.subcore_barrier()

    def blk(b, _):
        base = wid * RW + b * RB
        pltpu.sync_copy(index.at[pl.ds(base, RB)], idx_v)
        pltpu.sync_copy(ev.at[pl.ds(3 * base, 3 * RB)], ev_v)
        pltpu.sync_copy(er.at[pl.ds(3 * base, 3 * RB)], er_v)
        _build_idx3(idx_v, i3_v, pats)
        pltpu.sync_copy(ev_v, acc_v.at[i3_v], add=True)
        pltpu.sync_copy(er_v, acc_r.at[i3_v], add=True)
        pltpu.sync_copy(ones_v, acc_c.at[idx_v], add=True)
        return 0

    lax.fori_loop(0, NB, blk, 0)
    plsc.subcore_barrier()

    pltpu.sync_copy(acc_v.at[pl.ds(s * ZL, ZL)], zb_v)
    pltpu.sync_copy(zb_v, psum_v.at[c, pl.ds(s * ZL, ZL)])
    pltpu.sync_copy(acc_r.at[pl.ds(s * ZL, ZL)], zb_v)
    pltpu.sync_copy(zb_v, psum_r.at[c, pl.ds(s * ZL, ZL)])
    pltpu.sync_copy(acc_c.at[pl.ds(s * CL, CL)], zb_v.at[pl.ds(0, CL)])
    pltpu.sync_copy(zb_v.at[pl.ds(0, CL)], pcnt.at[c, pl.ds(s * CL, CL)])


def _means_body(psum_v, psum_r, pcnt, mean_v, mean_r, cnt_tot,
                a_v, b_v, cn_v, rc_v, m_v):
    wid, c, s = _wid()
    base = wid * SWL
    pats = _patterns()

    pltpu.sync_copy(pcnt.at[0, pl.ds(base, SWL)], cn_v)
    pltpu.sync_copy(pcnt.at[1, pl.ds(base, SWL)], rc_v)

    def cbody(k, _):
        tot = cn_v[pl.ds(16 * k, 16)] + rc_v[pl.ds(16 * k, 16)]
        cn_v[pl.ds(16 * k, 16)] = tot
        rc_v[pl.ds(16 * k, 16)] = 1.0 / jnp.maximum(tot, 1.0)
        return 0

    lax.fori_loop(0, SWL // 16, cbody, 0)
    pltpu.sync_copy(cn_v, cnt_tot.at[pl.ds(base, SWL)])

    for src, dst in ((psum_v, mean_v), (psum_r, mean_r)):
        pltpu.sync_copy(src.at[0, pl.ds(3 * base, 3 * SWL)], a_v)
        pltpu.sync_copy(src.at[1, pl.ds(3 * base, 3 * SWL)], b_v)

        def mbody(k, _):
            for v, (rp, cp) in enumerate(pats):
                o = 48 * k + 16 * v
                rc = plsc.load_gather(rc_v, [rp + 16 * k])
                m_v[pl.ds(o, 16)] = (
                    a_v[pl.ds(o, 16)] + b_v[pl.ds(o, 16)]) * rc
            return 0

        lax.fori_loop(0, SWL // 16, mbody, 0)
        pltpu.sync_copy(m_v, dst.at[pl.ds(3 * base, 3 * SWL)])


def _main_body(t, index, v0, ev, er, mean_v, mean_r,
               v_t, evc, erc, r_pre, psum_rt,
               idx_v, i3_v, t_v, v0_v, ev_v, er_v, mv_v, mr_v,
               vt_v, evc_v, erc_v, rp_v, al_v, sg_v, co_v, sr_v, zb_v,
               acc_rt):
    wid, c, s = _wid()
    pats = _patterns()

    _zero_fill(zb_v, ZL)
    pltpu.sync_copy(zb_v, acc_rt.at[pl.ds(s * ZL, ZL)])
    plsc.subcore_barrier()

    def blk(b, _):
        base = wid * RW + b * RB
        pltpu.sync_copy(index.at[pl.ds(base, RB)], idx_v)
        pltpu.sync_copy(t.at[pl.ds(base, RB)], t_v)
        pltpu.sync_copy(v0.at[pl.ds(3 * base, 3 * RB)], v0_v)
        pltpu.sync_copy(ev.at[pl.ds(3 * base, 3 * RB)], ev_v)
        pltpu.sync_copy(er.at[pl.ds(3 * base, 3 * RB)], er_v)
        _build_idx3(idx_v, i3_v, pats)
        pltpu.sync_copy(mean_v.at[i3_v], mv_v)
        pltpu.sync_copy(mean_r.at[i3_v], mr_v)

        def coef(k, _):
            ts = T_SCALE * t_v[pl.ds(16 * k, 16)]
            e = jnp.exp(-ts)
            al_v[pl.ds(16 * k, 16)] = e
            sg_v[pl.ds(16 * k, 16)] = _sqrt16(
                jnp.maximum(1.0 - e * e, EPS))
            co_v[pl.ds(16 * k, 16)] = (1.0 - e) / (1.0 + e)
            sr_v[pl.ds(16 * k, 16)] = _sqrt16(
                jnp.maximum(2.0 * ts + 8.0 * e / (1.0 + e) - 4.0, EPS))
            return 0

        lax.fori_loop(0, MB, coef, 0)

        def comb(k, _):
            for v, (rp, cp) in enumerate(pats):
                o = 48 * k + 16 * v
                rows = rp + 16 * k
                ec = ev_v[pl.ds(o, 16)] - mv_v[pl.ds(o, 16)]
                evc_v[pl.ds(o, 16)] = ec
                rc = er_v[pl.ds(o, 16)] - mr_v[pl.ds(o, 16)]
                erc_v[pl.ds(o, 16)] = rc
                v0x = v0_v[pl.ds(o, 16)]
                al = plsc.load_gather(al_v, [rows])
                sg = plsc.load_gather(sg_v, [rows])
                vt = al * v0x + sg * ec
                vt_v[pl.ds(o, 16)] = vt
                co = plsc.load_gather(co_v, [rows])
                sr = plsc.load_gather(sr_v, [rows])
                rp_v[pl.ds(o, 16)] = _wrap_signed16(
                    co * (vt + v0x) + sr * rc)
            return 0

        lax.fori_loop(0, MB, comb, 0)

        pltpu.sync_copy(vt_v.at[pl.ds(0, 3 * RB)], v_t.at[pl.ds(3 * base, 3 * RB)])
        pltpu.sync_copy(evc_v, evc.at[pl.ds(3 * base, 3 * RB)])
        pltpu.sync_copy(erc_v, erc.at[pl.ds(3 * base, 3 * RB)])
        pltpu.sync_copy(rp_v, r_pre.at[pl.ds(3 * base, 3 * RB)])
        pltpu.sync_copy(rp_v, acc_rt.at[i3_v], add=True)
        return 0

    lax.fori_loop(0, NB, blk, 0)
    plsc.subcore_barrier()

    pltpu.sync_copy(acc_rt.at[pl.ds(s * ZL, ZL)], zb_v)
    pltpu.sync_copy(zb_v, psum_rt.at[c, pl.ds(s * ZL, ZL)])


def _rt_means_body(psum_rt, cnt_tot, mean_rt, a_v, b_v, rc_v, m_v):
    wid, c, s = _wid()
    base = wid * SWL
    pats = _patterns()

    pltpu.sync_copy(cnt_tot.at[pl.ds(base, SWL)], rc_v)

    def cbody(k, _):
        rc_v[pl.ds(16 * k, 16)] = 1.0 / jnp.maximum(
            rc_v[pl.ds(16 * k, 16)], 1.0)
        return 0

    lax.fori_loop(0, SWL // 16, cbody, 0)

    pltpu.sync_copy(psum_rt.at[0, pl.ds(3 * base, 3 * SWL)], a_v)
    pltpu.sync_copy(psum_rt.at[1, pl.ds(3 * base, 3 * SWL)], b_v)

    def mbody(k, _):
        for v, (rp, cp) in enumerate(pats):
            o = 48 * k + 16 * v
            rc = plsc.load_gather(rc_v, [rp + 16 * k])
            m_v[pl.ds(o, 16)] = (a_v[pl.ds(o, 16)] + b_v[pl.ds(o, 16)]) * rc
        return 0

    lax.fori_loop(0, SWL // 16, mbody, 0)
    pltpu.sync_copy(m_v, mean_rt.at[pl.ds(3 * base, 3 * SWL)])


def _final_body(f0, r_pre, index, mean_rt, r_t, f_t,
                idx_v, i3_v, f0_v, rp_v, mrt_v, rt_v, ft_v):
    wid, c, s = _wid()
    pats = _patterns()

    def blk(b, _):
        base = wid * RW + b * RB
        pltpu.sync_copy(index.at[pl.ds(base, RB)], idx_v)
        pltpu.sync_copy(f0.at[pl.ds(3 * base, 3 * RB)], f0_v)
        pltpu.sync_copy(r_pre.at[pl.ds(3 * base, 3 * RB)], rp_v)
        _build_idx3(idx_v, i3_v, pats)
        pltpu.sync_copy(mean_rt.at[i3_v], mrt_v)

        def comb(k, _):
            for v in range(3):
                o = 48 * k + 16 * v
                rt = _wrap_signed16(rp_v[pl.ds(o, 16)] - mrt_v[pl.ds(o, 16)])
                rt_v[pl.ds(o, 16)] = rt
                ft_v[pl.ds(o, 16)] = _wrap_frac16(f0_v[pl.ds(o, 16)] + rt)
            return 0

        lax.fori_loop(0, MB, comb, 0)
        pltpu.sync_copy(rt_v, r_t.at[pl.ds(3 * base, 3 * RB)])
        pltpu.sync_copy(ft_v, f_t.at[pl.ds(3 * base, 3 * RB)])
        return 0

    lax.fori_loop(0, NB, blk, 0)


def _f32(*shape):
    return jax.ShapeDtypeStruct(shape, jnp.float32)


def kernel(t, f0, index, v0, epsilon_v, epsilon_r):
    f0f = f0.reshape(-1)
    v0f = v0.reshape(-1)
    evf = epsilon_v.reshape(-1)
    erf = epsilon_r.reshape(-1)

    sums = pl.kernel(
        _sums_body,
        out_type=(_f32(NC, 3 * S), _f32(NC, 3 * S), _f32(NC, S)),
        mesh=_mesh(),
        compiler_params=_params,
        scratch_types=[
            pltpu.VMEM((RB,), jnp.int32),
            pltpu.VMEM((3 * RB,), jnp.int32),
            pltpu.VMEM((3 * RB,), jnp.float32),
            pltpu.VMEM((3 * RB,), jnp.float32),
            pltpu.VMEM((RB,), jnp.float32),
            pltpu.VMEM((ZL,), jnp.float32),
            pltpu.VMEM_SHARED((3 * S,), jnp.float32),
            pltpu.VMEM_SHARED((3 * S,), jnp.float32),
            pltpu.VMEM_SHARED((S,), jnp.float32),
        ],
    )
    psum_v, psum_r, pcnt = sums(index, evf, erf)

    means = pl.kernel(
        _means_body,
        out_type=(_f32(3 * S), _f32(3 * S), _f32(S)),
        mesh=_mesh(),
        compiler_params=_params,
        scratch_types=[
            pltpu.VMEM((3 * SWL,), jnp.float32),
            pltpu.VMEM((3 * SWL,), jnp.float32),
            pltpu.VMEM((SWL,), jnp.float32),
            pltpu.VMEM((SWL,), jnp.float32),
            pltpu.VMEM((3 * SWL,), jnp.float32),
        ],
    )
    mean_v, mean_r, cnt_tot = means(psum_v, psum_r, pcnt)

    main = pl.kernel(
        _main_body,
        out_type=(_f32(3 * N), _f32(3 * N), _f32(3 * N), _f32(3 * N),
                  _f32(NC, 3 * S)),
        mesh=_mesh(),
        compiler_params=_params,
        scratch_types=[
            pltpu.VMEM((RB,), jnp.int32),
            pltpu.VMEM((3 * RB,), jnp.int32),
            pltpu.VMEM((RB,), jnp.float32),
        ] + [pltpu.VMEM((3 * RB,), jnp.float32) for _ in range(9)] + [
            pltpu.VMEM((RB,), jnp.float32) for _ in range(4)
        ] + [pltpu.VMEM((ZL,), jnp.float32),
             pltpu.VMEM_SHARED((3 * S,), jnp.float32)],
    )
    v_t, evc, erc, r_pre, psum_rt = main(
        t, index, v0f, evf, erf, mean_v, mean_r)

    rt_means = pl.kernel(
        _rt_means_body,
        out_type=_f32(3 * S),
        mesh=_mesh(),
        compiler_params=_params,
        scratch_types=[
            pltpu.VMEM((3 * SWL,), jnp.float32),
            pltpu.VMEM((3 * SWL,), jnp.float32),
            pltpu.VMEM((SWL,), jnp.float32),
            pltpu.VMEM((3 * SWL,), jnp.float32),
        ],
    )
    mean_rt = rt_means(psum_rt, cnt_tot)

    final = pl.kernel(
        _final_body,
        out_type=(_f32(3 * N), _f32(3 * N)),
        mesh=_mesh(),
        compiler_params=_params,
        scratch_types=[
            pltpu.VMEM((RB,), jnp.int32),
            pltpu.VMEM((3 * RB,), jnp.int32),
        ] + [pltpu.VMEM((3 * RB,), jnp.float32) for _ in range(5)],
    )
    r_t, f_t = final(f0f, r_pre, index, mean_rt)

    shape = (N, 3)
    return (f_t.reshape(shape), v_t.reshape(shape), evc.reshape(shape),
            erc.reshape(shape), r_t.reshape(shape))


# speedup vs baseline: 1.1737x; 1.0658x over previous
import functools

import jax
import jax.numpy as jnp
from jax import lax
from jax.experimental import pallas as pl
from jax.experimental.pallas import tpu as pltpu
from jax.experimental.pallas import tpu_sc as plsc

N = 1600000
S = 32768
EPS = 1e-05
T_SCALE = 2.0

NC = 2
NS = 16
NW = NC * NS
RW = N // NW
RB = 2000
NB = RW // RB
MB = RB // 16
SWL = S // NW
ZL = 3 * S // NS
CL = S // NS

_params = pltpu.CompilerParams(needs_layout_passes=False)

_mesh = functools.partial(
    plsc.VectorSubcoreMesh, core_axis_name="c", subcore_axis_name="s",
    num_cores=NC, num_subcores=NS)


def _wid():
    c = lax.axis_index("c")
    s = lax.axis_index("s")
    return s * NC + c, c, s


def _patterns():
    i = lax.iota(jnp.int32, 16)
    pats = []
    for v in range(3):
        flat = i + 16 * v
        pats.append((flat // 3, flat % 3))
    return pats


def _zero_fill(ref, n):
    z = jnp.zeros((16,), jnp.float32)

    def body(k, _):
        ref[pl.ds(16 * k, 16)] = z
        return 0

    lax.fori_loop(0, n // 16, body, 0)


def _build_idx3(idx_v, ioff, i3_v, pats):
    def body(k, _):
        for v, (rp, cp) in enumerate(pats):
            seg = plsc.load_gather(idx_v, [ioff + rp + 16 * k])
            i3_v[pl.ds(48 * k + 16 * v, 16)] = seg * 3 + cp
        return 0

    lax.fori_loop(0, MB, body, 0)


def _sqrt16(x):
    y = plsc.bitcast(
        jnp.int32(0x5F3759DF) - (plsc.bitcast(x, jnp.int32) >> 1), jnp.float32)
    half = x * 0.5
    for _ in range(3):
        y = y * (1.5 - half * y * y)
    return x * y


def _floor16(x):
    t = lax.convert_element_type(
        lax.convert_element_type(x, jnp.int32), jnp.float32)
    return jnp.where(t > x, t - 1.0, t)


def _wrap_signed16(x):
    y = x + 0.5
    return (y - _floor16(y)) - 0.5


def _wrap_frac16(x):
    return x - _floor16(x)


def _sums_body(index, ev, er, psum_v, psum_r, pcnt,
               idx_v, i3_v, ev_v, er_v, ones_v, zb_v, acc_v, acc_r, acc_c,
               six0, six1, sev0, sev1, ser0, ser1):
    wid, c, s = _wid()
    pats = _patterns()
    sems = ((six0, sev0, ser0), (six1, sev1, ser1))

    one = jnp.full((16,), 1.0, jnp.float32)

    def ones_body(k, _):
        ones_v[pl.ds(16 * k, 16)] = one
        return 0

    lax.fori_loop(0, 3 * RB // 16, ones_body, 0)

    _zero_fill(zb_v, ZL)
    pltpu.sync_copy(zb_v, acc_v.at[pl.ds(s * ZL, ZL)])
    pltpu.sync_copy(zb_v, acc_r.at[pl.ds(s * ZL, ZL)])
    pltpu.sync_copy(zb_v, acc_c.at[pl.ds(s * ZL, ZL)])
    plsc.subcore_barrier()

    def _dmas(b, par):
        base = wid * RW + b * RB
        six, sev, ser = sems[par]
        return (
            pltpu.make_async_copy(index.at[pl.ds(base, RB)],
                                  idx_v.at[pl.ds(par * RB, RB)], six),
            pltpu.make_async_copy(ev.at[pl.ds(3 * base, 3 * RB)],
                                  ev_v.at[pl.ds(par * 3 * RB, 3 * RB)], sev),
            pltpu.make_async_copy(er.at[pl.ds(3 * base, 3 * RB)],
                                  er_v.at[pl.ds(par * 3 * RB, 3 * RB)], ser),
        )

    def _start(b, par):
        for d in _dmas(b, par):
            d.start()

    def _wait(b, par):
        for d in _dmas(b, par):
            d.wait()

    _start(0, 0)

    def blk(b, _):
        par = lax.rem(b, 2)
        nb_ok = b + 1 < NB

        @pl.when(jnp.logical_and(nb_ok, par == 0))
        def _():
            _start(b + 1, 1)

        @pl.when(jnp.logical_and(nb_ok, par == 1))
        def _():
            _start(b + 1, 0)

        @pl.when(par == 0)
        def _():
            _wait(b, 0)

        @pl.when(par == 1)
        def _():
            _wait(b, 1)

        po = par * RB
        po3 = par * 3 * RB
        _build_idx3(idx_v, po, i3_v, pats)
        pltpu.sync_copy(ev_v.at[pl.ds(po3, 3 * RB)], acc_v.at[i3_v], add=True)
        pltpu.sync_copy(er_v.at[pl.ds(po3, 3 * RB)], acc_r.at[i3_v], add=True)
        pltpu.sync_copy(ones_v, acc_c.at[i3_v], add=True)
        return 0

    lax.fori_loop(0, NB, blk, 0)
    plsc.subcore_barrier()

    pltpu.sync_copy(acc_v.at[pl.ds(s * ZL, ZL)], zb_v)
    pltpu.sync_copy(zb_v, psum_v.at[c, pl.ds(s * ZL, ZL)])
    pltpu.sync_copy(acc_r.at[pl.ds(s * ZL, ZL)], zb_v)
    pltpu.sync_copy(zb_v, psum_r.at[c, pl.ds(s * ZL, ZL)])
    pltpu.sync_copy(acc_c.at[pl.ds(s * ZL, ZL)], zb_v)
    pltpu.sync_copy(zb_v, pcnt.at[c, pl.ds(s * ZL, ZL)])


def _means_body(psum_v, psum_r, pcnt, mean_v, mean_r, cnt_tot,
                a_v, b_v, cn_v, rc_v, m_v):
    wid, c, s = _wid()
    base = wid * SWL
    pats = _patterns()

    pltpu.sync_copy(pcnt.at[0, pl.ds(3 * base, 3 * SWL)], a_v)
    pltpu.sync_copy(pcnt.at[1, pl.ds(3 * base, 3 * SWL)], b_v)
    i16 = lax.iota(jnp.int32, 16)

    def cbody(k, _):
        i3 = (i16 + 16 * k) * 3
        tot = plsc.load_gather(a_v, [i3]) + plsc.load_gather(b_v, [i3])
        cn_v[pl.ds(16 * k, 16)] = tot
        rc_v[pl.ds(16 * k, 16)] = 1.0 / jnp.maximum(tot, 1.0)
        return 0

    lax.fori_loop(0, SWL // 16, cbody, 0)
    pltpu.sync_copy(cn_v, cnt_tot.at[pl.ds(base, SWL)])

    for src, dst in ((psum_v, mean_v), (psum_r, mean_r)):
        pltpu.sync_copy(src.at[0, pl.ds(3 * base, 3 * SWL)], a_v)
        pltpu.sync_copy(src.at[1, pl.ds(3 * base, 3 * SWL)], b_v)

        def mbody(k, _):
            for v, (rp, cp) in enumerate(pats):
                o = 48 * k + 16 * v
                rc = plsc.load_gather(rc_v, [rp + 16 * k])
                m_v[pl.ds(o, 16)] = (
                    a_v[pl.ds(o, 16)] + b_v[pl.ds(o, 16)]) * rc
            return 0

        lax.fori_loop(0, SWL // 16, mbody, 0)
        pltpu.sync_copy(m_v, dst.at[pl.ds(3 * base, 3 * SWL)])


def _main_body(t, index, v0, ev, er, mean_v, mean_r,
               v_t, evc, erc, r_pre, psum_rt,
               idx_v, i3_v, t_v, v0_v, ev_v, er_v, mv_v, mr_v,
               vt_v, evc_v, erc_v, rp_v, al_v, sg_v, co_v, sr_v, zb_v,
               acc_rt, shv, shr,
               six0, six1, st0, st1, sv00, sv01, sev0, sev1, ser0, ser1,
               sgv, sgr, sout):
    wid, c, s = _wid()
    pats = _patterns()
    sems = ((six0, st0, sv00, sev0, ser0), (six1, st1, sv01, sev1, ser1))

    _zero_fill(zb_v, ZL)
    pltpu.sync_copy(zb_v, acc_rt.at[pl.ds(s * ZL, ZL)])
    pltpu.sync_copy(mean_v.at[pl.ds(s * ZL, ZL)], zb_v)
    pltpu.sync_copy(zb_v, shv.at[pl.ds(s * ZL, ZL)])
    pltpu.sync_copy(mean_r.at[pl.ds(s * ZL, ZL)], zb_v)
    pltpu.sync_copy(zb_v, shr.at[pl.ds(s * ZL, ZL)])
    plsc.subcore_barrier()

    def _dmas(b, par):
        base = wid * RW + b * RB
        six, st, sv0, sev, ser = sems[par]
        return (
            pltpu.make_async_copy(index.at[pl.ds(base, RB)],
                                  idx_v.at[pl.ds(par * RB, RB)], six),
            pltpu.make_async_copy(t.at[pl.ds(base, RB)],
                                  t_v.at[pl.ds(par * RB, RB)], st),
            pltpu.make_async_copy(v0.at[pl.ds(3 * base, 3 * RB)],
                                  v0_v.at[pl.ds(par * 3 * RB, 3 * RB)], sv0),
            pltpu.make_async_copy(ev.at[pl.ds(3 * base, 3 * RB)],
                                  ev_v.at[pl.ds(par * 3 * RB, 3 * RB)], sev),
            pltpu.make_async_copy(er.at[pl.ds(3 * base, 3 * RB)],
                                  er_v.at[pl.ds(par * 3 * RB, 3 * RB)], ser),
        )

    def _start(b, par):
        for d in _dmas(b, par):
            d.start()

    def _wait(b, par):
        for d in _dmas(b, par):
            d.wait()

    _start(0, 0)

    def blk(b, _):
        par = lax.rem(b, 2)
        nb_ok = b + 1 < NB

        @pl.when(jnp.logical_and(nb_ok, par == 0))
        def _():
            _start(b + 1, 1)

        @pl.when(jnp.logical_and(nb_ok, par == 1))
        def _():
            _start(b + 1, 0)

        @pl.when(par == 0)
        def _():
            _wait(b, 0)

        @pl.when(par == 1)
        def _():
            _wait(b, 1)

        po = par * RB
        po3 = par * 3 * RB
        base = wid * RW + b * RB
        _build_idx3(idx_v, po, i3_v, pats)
        gv = pltpu.async_copy(shv.at[i3_v], mv_v, sgv)
        gr = pltpu.async_copy(shr.at[i3_v], mr_v, sgr)

        def coef(k, _):
            ts = T_SCALE * t_v[pl.ds(po + 16 * k, 16)]
            e = jnp.exp(-ts)
            al_v[pl.ds(16 * k, 16)] = e
            sg_v[pl.ds(16 * k, 16)] = _sqrt16(
                jnp.maximum(1.0 - e * e, EPS))
            co_v[pl.ds(16 * k, 16)] = (1.0 - e) / (1.0 + e)
            sr_v[pl.ds(16 * k, 16)] = _sqrt16(
                jnp.maximum(2.0 * ts + 8.0 * e / (1.0 + e) - 4.0, EPS))
            return 0

        lax.fori_loop(0, MB, coef, 0)
        gv.wait()
        gr.wait()

        def comb(k, _):
            for v, (rp, cp) in enumerate(pats):
                o = 48 * k + 16 * v
                rows = rp + 16 * k
                ec = ev_v[pl.ds(po3 + o, 16)] - mv_v[pl.ds(o, 16)]
                evc_v[pl.ds(o, 16)] = ec
                rc = er_v[pl.ds(po3 + o, 16)] - mr_v[pl.ds(o, 16)]
                erc_v[pl.ds(o, 16)] = rc
                v0x = v0_v[pl.ds(po3 + o, 16)]
                al = plsc.load_gather(al_v, [rows])
                sg = plsc.load_gather(sg_v, [rows])
                vt = al * v0x + sg * ec
                vt_v[pl.ds(o, 16)] = vt
                co = plsc.load_gather(co_v, [rows])
                sr = plsc.load_gather(sr_v, [rows])
                rp_v[pl.ds(o, 16)] = _wrap_signed16(
                    co * (vt + v0x) + sr * rc)
            return 0

        lax.fori_loop(0, MB, comb, 0)

        outs = (
            pltpu.async_copy(vt_v, v_t.at[pl.ds(3 * base, 3 * RB)], sout),
            pltpu.async_copy(evc_v, evc.at[pl.ds(3 * base, 3 * RB)], sout),
            pltpu.async_copy(erc_v, erc.at[pl.ds(3 * base, 3 * RB)], sout),
            pltpu.async_copy(rp_v, r_pre.at[pl.ds(3 * base, 3 * RB)], sout),
        )
        pltpu.sync_copy(rp_v, acc_rt.at[i3_v], add=True)
        for d in outs:
            d.wait()
        return 0

    lax.fori_loop(0, NB, blk, 0)
    plsc.subcore_barrier()

    pltpu.sync_copy(acc_rt.at[pl.ds(s * ZL, ZL)], zb_v)
    pltpu.sync_copy(zb_v, psum_rt.at[c, pl.ds(s * ZL, ZL)])


def _rt_means_body(psum_rt, cnt_tot, mean_rt, a_v, b_v, rc_v, m_v):
    wid, c, s = _wid()
    base = wid * SWL
    pats = _patterns()

    pltpu.sync_copy(cnt_tot.at[pl.ds(base, SWL)], rc_v)

    def cbody(k, _):
        rc_v[pl.ds(16 * k, 16)] = 1.0 / jnp.maximum(
            rc_v[pl.ds(16 * k, 16)], 1.0)
        return 0

    lax.fori_loop(0, SWL // 16, cbody, 0)

    pltpu.sync_copy(psum_rt.at[0, pl.ds(3 * base, 3 * SWL)], a_v)
    pltpu.sync_copy(psum_rt.at[1, pl.ds(3 * base, 3 * SWL)], b_v)

    def mbody(k, _):
        for v, (rp, cp) in enumerate(pats):
            o = 48 * k + 16 * v
            rc = plsc.load_gather(rc_v, [rp + 16 * k])
            m_v[pl.ds(o, 16)] = (a_v[pl.ds(o, 16)] + b_v[pl.ds(o, 16)]) * rc
        return 0

    lax.fori_loop(0, SWL // 16, mbody, 0)
    pltpu.sync_copy(m_v, mean_rt.at[pl.ds(3 * base, 3 * SWL)])


def _final_body(f0, r_pre, index, mean_rt, r_t, f_t,
                idx_v, i3_v, f0_v, rp_v, mrt_v, rt_v, ft_v, zb_v, shm,
                six0, six1, sf0, sf1, sr0, sr1, sout):
    wid, c, s = _wid()
    pats = _patterns()
    sems = ((six0, sf0, sr0), (six1, sf1, sr1))

    pltpu.sync_copy(mean_rt.at[pl.ds(s * ZL, ZL)], zb_v)
    pltpu.sync_copy(zb_v, shm.at[pl.ds(s * ZL, ZL)])
    plsc.subcore_barrier()

    def _dmas(b, par):
        base = wid * RW + b * RB
        six, sf, sr = sems[par]
        return (
            pltpu.make_async_copy(index.at[pl.ds(base, RB)],
                                  idx_v.at[pl.ds(par * RB, RB)], six),
            pltpu.make_async_copy(f0.at[pl.ds(3 * base, 3 * RB)],
                                  f0_v.at[pl.ds(par * 3 * RB, 3 * RB)], sf),
            pltpu.make_async_copy(r_pre.at[pl.ds(3 * base, 3 * RB)],
                                  rp_v.at[pl.ds(par * 3 * RB, 3 * RB)], sr),
        )

    def _start(b, par):
        for d in _dmas(b, par):
            d.start()

    def _wait(b, par):
        for d in _dmas(b, par):
            d.wait()

    _start(0, 0)

    def blk(b, _):
        par = lax.rem(b, 2)
        nb_ok = b + 1 < NB

        @pl.when(jnp.logical_and(nb_ok, par == 0))
        def _():
            _start(b + 1, 1)

        @pl.when(jnp.logical_and(nb_ok, par == 1))
        def _():
            _start(b + 1, 0)

        @pl.when(par == 0)
        def _():
            _wait(b, 0)

        @pl.when(par == 1)
        def _():
            _wait(b, 1)

        po3 = par * 3 * RB
        base = wid * RW + b * RB
        _build_idx3(idx_v, par * RB, i3_v, pats)
        pltpu.sync_copy(shm.at[i3_v], mrt_v)

        def comb(k, _):
            for v in range(3):
                o = 48 * k + 16 * v
                rt = _wrap_signed16(
                    rp_v[pl.ds(po3 + o, 16)] - mrt_v[pl.ds(o, 16)])
                rt_v[pl.ds(o, 16)] = rt
                ft_v[pl.ds(o, 16)] = _wrap_frac16(
                    f0_v[pl.ds(po3 + o, 16)] + rt)
            return 0

        lax.fori_loop(0, MB, comb, 0)
        outs = (
            pltpu.async_copy(rt_v, r_t.at[pl.ds(3 * base, 3 * RB)], sout),
            pltpu.async_copy(ft_v, f_t.at[pl.ds(3 * base, 3 * RB)], sout),
        )
        for d in outs:
            d.wait()
        return 0

    lax.fori_loop(0, NB, blk, 0)


def _f32(*shape):
    return jax.ShapeDtypeStruct(shape, jnp.float32)


def kernel(t, f0, index, v0, epsilon_v, epsilon_r):
    f0f = f0.reshape(-1)
    v0f = v0.reshape(-1)
    evf = epsilon_v.reshape(-1)
    erf = epsilon_r.reshape(-1)

    sums = pl.kernel(
        _sums_body,
        out_type=(_f32(NC, 3 * S), _f32(NC, 3 * S), _f32(NC, 3 * S)),
        mesh=_mesh(),
        compiler_params=_params,
        scratch_types=[
            pltpu.VMEM((2 * RB,), jnp.int32),
            pltpu.VMEM((3 * RB,), jnp.int32),
            pltpu.VMEM((2 * 3 * RB,), jnp.float32),
            pltpu.VMEM((2 * 3 * RB,), jnp.float32),
            pltpu.VMEM((3 * RB,), jnp.float32),
            pltpu.VMEM((ZL,), jnp.float32),
            pltpu.VMEM_SHARED((3 * S,), jnp.float32),
            pltpu.VMEM_SHARED((3 * S,), jnp.float32),
            pltpu.VMEM_SHARED((3 * S,), jnp.float32),
        ] + [pltpu.SemaphoreType.DMA] * 6,
    )
    psum_v, psum_r, pcnt = sums(index, evf, erf)

    means = pl.kernel(
        _means_body,
        out_type=(_f32(3 * S), _f32(3 * S), _f32(S)),
        mesh=_mesh(),
        compiler_params=_params,
        scratch_types=[
            pltpu.VMEM((3 * SWL,), jnp.float32),
            pltpu.VMEM((3 * SWL,), jnp.float32),
            pltpu.VMEM((SWL,), jnp.float32),
            pltpu.VMEM((SWL,), jnp.float32),
            pltpu.VMEM((3 * SWL,), jnp.float32),
        ],
    )
    mean_v, mean_r, cnt_tot = means(psum_v, psum_r, pcnt)

    main = pl.kernel(
        _main_body,
        out_type=(_f32(3 * N), _f32(3 * N), _f32(3 * N), _f32(3 * N),
                  _f32(NC, 3 * S)),
        mesh=_mesh(),
        compiler_params=_params,
        scratch_types=[
            pltpu.VMEM((2 * RB,), jnp.int32),
            pltpu.VMEM((3 * RB,), jnp.int32),
            pltpu.VMEM((2 * RB,), jnp.float32),
            pltpu.VMEM((2 * 3 * RB,), jnp.float32),
            pltpu.VMEM((2 * 3 * RB,), jnp.float32),
            pltpu.VMEM((2 * 3 * RB,), jnp.float32),
            pltpu.VMEM((3 * RB,), jnp.float32),
            pltpu.VMEM((3 * RB,), jnp.float32),
            pltpu.VMEM((3 * RB,), jnp.float32),
            pltpu.VMEM((3 * RB,), jnp.float32),
            pltpu.VMEM((3 * RB,), jnp.float32),
            pltpu.VMEM((3 * RB,), jnp.float32),
            pltpu.VMEM((RB,), jnp.float32),
            pltpu.VMEM((RB,), jnp.float32),
            pltpu.VMEM((RB,), jnp.float32),
            pltpu.VMEM((RB,), jnp.float32),
            pltpu.VMEM((ZL,), jnp.float32),
            pltpu.VMEM_SHARED((3 * S,), jnp.float32),
            pltpu.VMEM_SHARED((3 * S,), jnp.float32),
            pltpu.VMEM_SHARED((3 * S,), jnp.float32),
        ] + [pltpu.SemaphoreType.DMA] * 13,
    )
    v_t, evc, erc, r_pre, psum_rt = main(
        t, index, v0f, evf, erf, mean_v, mean_r)

    rt_means = pl.kernel(
        _rt_means_body,
        out_type=_f32(3 * S),
        mesh=_mesh(),
        compiler_params=_params,
        scratch_types=[
            pltpu.VMEM((3 * SWL,), jnp.float32),
            pltpu.VMEM((3 * SWL,), jnp.float32),
            pltpu.VMEM((SWL,), jnp.float32),
            pltpu.VMEM((3 * SWL,), jnp.float32),
        ],
    )
    mean_rt = rt_means(psum_rt, cnt_tot)

    final = pl.kernel(
        _final_body,
        out_type=(_f32(3 * N), _f32(3 * N)),
        mesh=_mesh(),
        compiler_params=_params,
        scratch_types=[
            pltpu.VMEM((2 * RB,), jnp.int32),
            pltpu.VMEM((3 * RB,), jnp.int32),
            pltpu.VMEM((2 * 3 * RB,), jnp.float32),
            pltpu.VMEM((2 * 3 * RB,), jnp.float32),
            pltpu.VMEM((3 * RB,), jnp.float32),
            pltpu.VMEM((3 * RB,), jnp.float32),
            pltpu.VMEM((3 * RB,), jnp.float32),
            pltpu.VMEM((ZL,), jnp.float32),
            pltpu.VMEM_SHARED((3 * S,), jnp.float32),
        ] + [pltpu.SemaphoreType.DMA] * 7,
    )
    r_t, f_t = final(f0f, r_pre, index, mean_rt)

    shape = (N, 3)
    return (f_t.reshape(shape), v_t.reshape(shape), evc.reshape(shape),
            erc.reshape(shape), r_t.reshape(shape))


# speedup vs baseline: 14.5141x; 12.3660x over previous
import functools

import jax
import jax.numpy as jnp
from jax import lax
from jax.experimental import pallas as pl
from jax.experimental.pallas import tpu as pltpu
from jax.experimental.pallas import tpu_sc as plsc

N = 1600000
S = 32768
EPS = 1e-05
T_SCALE = 2.0

NC = 2
NS = 16
NW = NC * NS
RW = N // NW
RB = 2000
NB = RW // RB
MB = RB // 16
SWL = S // NW
ZL = 3 * S // NS
CL = S // NS

_params = pltpu.CompilerParams(needs_layout_passes=False)

_mesh = functools.partial(
    plsc.VectorSubcoreMesh, core_axis_name="c", subcore_axis_name="s",
    num_cores=NC, num_subcores=NS)


def _wid():
    c = lax.axis_index("c")
    s = lax.axis_index("s")
    return s * NC + c, c, s


def _zero_fill(ref, n):
    z = jnp.zeros((16,), jnp.float32)

    def body(k, _):
        ref[pl.ds(16 * k, 16)] = z
        return 0

    lax.fori_loop(0, n // 16, body, 0)


def _build_i3(idx_v, ioff, i30, i31, i32):
    def body(k, _):
        seg = idx_v[pl.ds(ioff + 16 * k, 16)]
        i30[pl.ds(16 * k, 16)] = seg
        i31[pl.ds(16 * k, 16)] = seg + S
        i32[pl.ds(16 * k, 16)] = seg + 2 * S
        return 0

    lax.fori_loop(0, MB, body, 0)


def _sqrt16(x):
    y = plsc.bitcast(
        jnp.int32(0x5F3759DF) - (plsc.bitcast(x, jnp.int32) >> 1), jnp.float32)
    half = x * 0.5
    for _ in range(3):
        y = y * (1.5 - half * y * y)
    return x * y


def _floor16(x):
    t = lax.convert_element_type(
        lax.convert_element_type(x, jnp.int32), jnp.float32)
    return jnp.where(t > x, t - 1.0, t)


def _wrap_signed16(x):
    y = x + 0.5
    return (y - _floor16(y)) - 0.5


def _wrap_frac16(x):
    return x - _floor16(x)


def _par_branches(b, start, wait):
    par = lax.rem(b, 2)
    nb_ok = b + 1 < NB

    @pl.when(jnp.logical_and(nb_ok, par == 0))
    def _():
        start(b + 1, 1)

    @pl.when(jnp.logical_and(nb_ok, par == 1))
    def _():
        start(b + 1, 0)

    @pl.when(par == 0)
    def _():
        wait(b, 0)

    @pl.when(par == 1)
    def _():
        wait(b, 1)

    return par


def _sums_body(index, ev0, ev1, ev2, er0, er1, er2, psum_v, psum_r, pcnt,
               idx_v, i30, i31, i32, ev_v, er_v, ones_v, zb_v,
               acc_v, acc_r, acc_c, sin0, sin1):
    wid, c, s = _wid()
    sems = (sin0, sin1)
    evs = (ev0, ev1, ev2)
    ers = (er0, er1, er2)

    one = jnp.full((16,), 1.0, jnp.float32)

    def ones_body(k, _):
        ones_v[pl.ds(16 * k, 16)] = one
        return 0

    lax.fori_loop(0, RB // 16, ones_body, 0)

    _zero_fill(zb_v, ZL)
    pltpu.sync_copy(zb_v, acc_v.at[pl.ds(s * ZL, ZL)])
    pltpu.sync_copy(zb_v, acc_r.at[pl.ds(s * ZL, ZL)])
    pltpu.sync_copy(zb_v.at[pl.ds(0, CL)], acc_c.at[pl.ds(s * CL, CL)])
    plsc.subcore_barrier()

    def _dmas(b, par):
        base = wid * RW + b * RB
        sem = sems[par]
        ds = [pltpu.make_async_copy(index.at[pl.ds(base, RB)],
                                    idx_v.at[pl.ds(par * RB, RB)], sem)]
        for ci in range(3):
            ds.append(pltpu.make_async_copy(
                evs[ci].at[pl.ds(base, RB)],
                ev_v.at[pl.ds((par * 3 + ci) * RB, RB)], sem))
            ds.append(pltpu.make_async_copy(
                ers[ci].at[pl.ds(base, RB)],
                er_v.at[pl.ds((par * 3 + ci) * RB, RB)], sem))
        return ds

    def _start(b, par):
        for d in _dmas(b, par):
            d.start()

    def _wait(b, par):
        for d in _dmas(b, par):
            d.wait()

    _start(0, 0)

    def blk(b, _):
        par = _par_branches(b, _start, _wait)
        _build_i3(idx_v, par * RB, i30, i31, i32)
        i3s = (i30, i31, i32)
        for ci in range(3):
            pltpu.sync_copy(ev_v.at[pl.ds((par * 3 + ci) * RB, RB)],
                            acc_v.at[i3s[ci]], add=True)
            pltpu.sync_copy(er_v.at[pl.ds((par * 3 + ci) * RB, RB)],
                            acc_r.at[i3s[ci]], add=True)
        pltpu.sync_copy(ones_v, acc_c.at[i30], add=True)
        return 0

    lax.fori_loop(0, NB, blk, 0)
    plsc.subcore_barrier()

    pltpu.sync_copy(acc_v.at[pl.ds(s * ZL, ZL)], zb_v)
    pltpu.sync_copy(zb_v, psum_v.at[c, pl.ds(s * ZL, ZL)])
    pltpu.sync_copy(acc_r.at[pl.ds(s * ZL, ZL)], zb_v)
    pltpu.sync_copy(zb_v, psum_r.at[c, pl.ds(s * ZL, ZL)])
    pltpu.sync_copy(acc_c.at[pl.ds(s * CL, CL)], zb_v.at[pl.ds(0, CL)])
    pltpu.sync_copy(zb_v.at[pl.ds(0, CL)], pcnt.at[c, pl.ds(s * CL, CL)])


def _means_body(psum_v, psum_r, pcnt, mean_v, mean_r, cnt_tot,
                a_v, b_v, cn_v, rc_v, m_v):
    wid, c, s = _wid()
    base = wid * SWL
    pltpu.sync_copy(pcnt.at[0, pl.ds(base, SWL)], cn_v)
    pltpu.sync_copy(pcnt.at[1, pl.ds(base, SWL)], rc_v)

    def cbody(k, _):
        tot = cn_v[pl.ds(16 * k, 16)] + rc_v[pl.ds(16 * k, 16)]
        cn_v[pl.ds(16 * k, 16)] = tot
        rc_v[pl.ds(16 * k, 16)] = 1.0 / jnp.maximum(tot, 1.0)
        return 0

    lax.fori_loop(0, SWL // 16, cbody, 0)
    pltpu.sync_copy(cn_v, cnt_tot.at[pl.ds(base, SWL)])

    for src, dst in ((psum_v, mean_v), (psum_r, mean_r)):
        for ci in range(3):
            o = ci * S + base
            pltpu.sync_copy(src.at[0, pl.ds(o, SWL)], a_v)
            pltpu.sync_copy(src.at[1, pl.ds(o, SWL)], b_v)

            def mbody(k, _):
                m_v[pl.ds(16 * k, 16)] = (
                    a_v[pl.ds(16 * k, 16)] + b_v[pl.ds(16 * k, 16)]
                ) * rc_v[pl.ds(16 * k, 16)]
                return 0

            lax.fori_loop(0, SWL // 16, mbody, 0)
            pltpu.sync_copy(m_v, dst.at[pl.ds(o, SWL)])


def _main_body(t, index, v00, v01, v02, ev0, ev1, ev2, er0, er1, er2,
               mean_v, mean_r,
               vt0, vt1, vt2, ec0, ec1, ec2, rc0, rc1, rc2,
               rp0, rp1, rp2, psum_rt,
               idx_v, i30, i31, i32, t_v, v0_v, ev_v, er_v, mv_v, mr_v,
               vt_v, evc_v, erc_v, rp_v, al_v, sg_v, co_v, sr_v, zb_v,
               acc_rt, shv, shr, sin0, sin1, sgv, sgr, sout):
    wid, c, s = _wid()
    sems = (sin0, sin1)
    v0s = (v00, v01, v02)
    evs = (ev0, ev1, ev2)
    ers = (er0, er1, er2)
    vts = (vt0, vt1, vt2)
    ecs = (ec0, ec1, ec2)
    rcs = (rc0, rc1, rc2)
    rps = (rp0, rp1, rp2)

    _zero_fill(zb_v, ZL)
    pltpu.sync_copy(zb_v, acc_rt.at[pl.ds(s * ZL, ZL)])
    pltpu.sync_copy(mean_v.at[pl.ds(s * ZL, ZL)], zb_v)
    pltpu.sync_copy(zb_v, shv.at[pl.ds(s * ZL, ZL)])
    pltpu.sync_copy(mean_r.at[pl.ds(s * ZL, ZL)], zb_v)
    pltpu.sync_copy(zb_v, shr.at[pl.ds(s * ZL, ZL)])
    plsc.subcore_barrier()

    def _dmas(b, par):
        base = wid * RW + b * RB
        sem = sems[par]
        ds = [
            pltpu.make_async_copy(index.at[pl.ds(base, RB)],
                                  idx_v.at[pl.ds(par * RB, RB)], sem),
            pltpu.make_async_copy(t.at[pl.ds(base, RB)],
                                  t_v.at[pl.ds(par * RB, RB)], sem),
        ]
        for ci in range(3):
            po = (par * 3 + ci) * RB
            ds.append(pltpu.make_async_copy(
                v0s[ci].at[pl.ds(base, RB)], v0_v.at[pl.ds(po, RB)], sem))
            ds.append(pltpu.make_async_copy(
                evs[ci].at[pl.ds(base, RB)], ev_v.at[pl.ds(po, RB)], sem))
            ds.append(pltpu.make_async_copy(
                ers[ci].at[pl.ds(base, RB)], er_v.at[pl.ds(po, RB)], sem))
        return ds

    def _start(b, par):
        for d in _dmas(b, par):
            d.start()

    def _wait(b, par):
        for d in _dmas(b, par):
            d.wait()

    _start(0, 0)

    def blk(b, _):
        par = _par_branches(b, _start, _wait)
        po = par * RB
        base = wid * RW + b * RB
        _build_i3(idx_v, po, i30, i31, i32)
        i3s = (i30, i31, i32)
        gs = []
        for ci in range(3):
            gs.append(pltpu.async_copy(
                shv.at[i3s[ci]], mv_v.at[pl.ds(ci * RB, RB)], sgv))
            gs.append(pltpu.async_copy(
                shr.at[i3s[ci]], mr_v.at[pl.ds(ci * RB, RB)], sgr))

        def coef(k, _):
            ts = T_SCALE * t_v[pl.ds(po + 16 * k, 16)]
            e = jnp.exp(-ts)
            al_v[pl.ds(16 * k, 16)] = e
            sg_v[pl.ds(16 * k, 16)] = _sqrt16(jnp.maximum(1.0 - e * e, EPS))
            co_v[pl.ds(16 * k, 16)] = (1.0 - e) / (1.0 + e)
            sr_v[pl.ds(16 * k, 16)] = _sqrt16(
                jnp.maximum(2.0 * ts + 8.0 * e / (1.0 + e) - 4.0, EPS))
            return 0

        lax.fori_loop(0, MB, coef, 0)
        for g in gs:
            g.wait()

        def comb(k, _):
            o = 16 * k
            al = al_v[pl.ds(o, 16)]
            sg = sg_v[pl.ds(o, 16)]
            co = co_v[pl.ds(o, 16)]
            sr = sr_v[pl.ds(o, 16)]
            for ci in range(3):
                po3 = (par * 3 + ci) * RB + o
                oc = ci * RB + o
                ec = ev_v[pl.ds(po3, 16)] - mv_v[pl.ds(oc, 16)]
                evc_v[pl.ds(oc, 16)] = ec
                rc = er_v[pl.ds(po3, 16)] - mr_v[pl.ds(oc, 16)]
                erc_v[pl.ds(oc, 16)] = rc
                v0x = v0_v[pl.ds(po3, 16)]
                vt = al * v0x + sg * ec
                vt_v[pl.ds(oc, 16)] = vt
                rp_v[pl.ds(oc, 16)] = _wrap_signed16(
                    co * (vt + v0x) + sr * rc)
            return 0

        lax.fori_loop(0, MB, comb, 0)

        outs = []
        for ci in range(3):
            oc = pl.ds(ci * RB, RB)
            hs = pl.ds(base, RB)
            outs.append(pltpu.async_copy(vt_v.at[oc], vts[ci].at[hs], sout))
            outs.append(pltpu.async_copy(evc_v.at[oc], ecs[ci].at[hs], sout))
            outs.append(pltpu.async_copy(erc_v.at[oc], rcs[ci].at[hs], sout))
            outs.append(pltpu.async_copy(rp_v.at[oc], rps[ci].at[hs], sout))
            pltpu.sync_copy(rp_v.at[oc], acc_rt.at[i3s[ci]], add=True)
        for d in outs:
            d.wait()
        return 0

    lax.fori_loop(0, NB, blk, 0)
    plsc.subcore_barrier()

    pltpu.sync_copy(acc_rt.at[pl.ds(s * ZL, ZL)], zb_v)
    pltpu.sync_copy(zb_v, psum_rt.at[c, pl.ds(s * ZL, ZL)])


def _rt_means_body(psum_rt, cnt_tot, mean_rt, a_v, b_v, rc_v, m_v):
    wid, c, s = _wid()
    base = wid * SWL
    pltpu.sync_copy(cnt_tot.at[pl.ds(base, SWL)], rc_v)

    def cbody(k, _):
        rc_v[pl.ds(16 * k, 16)] = 1.0 / jnp.maximum(
            rc_v[pl.ds(16 * k, 16)], 1.0)
        return 0

    lax.fori_loop(0, SWL // 16, cbody, 0)

    for ci in range(3):
        o = ci * S + base
        pltpu.sync_copy(psum_rt.at[0, pl.ds(o, SWL)], a_v)
        pltpu.sync_copy(psum_rt.at[1, pl.ds(o, SWL)], b_v)

        def mbody(k, _):
            m_v[pl.ds(16 * k, 16)] = (
                a_v[pl.ds(16 * k, 16)] + b_v[pl.ds(16 * k, 16)]
            ) * rc_v[pl.ds(16 * k, 16)]
            return 0

        lax.fori_loop(0, SWL // 16, mbody, 0)
        pltpu.sync_copy(m_v, mean_rt.at[pl.ds(o, SWL)])


def _final_body(f00, f01, f02, rp0, rp1, rp2, index, mean_rt,
                rt0, rt1, rt2, ft0, ft1, ft2,
                idx_v, i30, i31, i32, f0_v, rp_v, mrt_v, rt_v, ft_v, zb_v,
                shm, sin0, sin1, sg, sout):
    wid, c, s = _wid()
    sems = (sin0, sin1)
    f0s = (f00, f01, f02)
    rps = (rp0, rp1, rp2)
    rts = (rt0, rt1, rt2)
    fts = (ft0, ft1, ft2)

    pltpu.sync_copy(mean_rt.at[pl.ds(s * ZL, ZL)], zb_v)
    pltpu.sync_copy(zb_v, shm.at[pl.ds(s * ZL, ZL)])
    plsc.subcore_barrier()

    def _dmas(b, par):
        base = wid * RW + b * RB
        sem = sems[par]
        ds = [pltpu.make_async_copy(index.at[pl.ds(base, RB)],
                                    idx_v.at[pl.ds(par * RB, RB)], sem)]
        for ci in range(3):
            po = (par * 3 + ci) * RB
            ds.append(pltpu.make_async_copy(
                f0s[ci].at[pl.ds(base, RB)], f0_v.at[pl.ds(po, RB)], sem))
            ds.append(pltpu.make_async_copy(
                rps[ci].at[pl.ds(base, RB)], rp_v.at[pl.ds(po, RB)], sem))
        return ds

    def _start(b, par):
        for d in _dmas(b, par):
            d.start()

    def _wait(b, par):
        for d in _dmas(b, par):
            d.wait()

    _start(0, 0)

    def blk(b, _):
        par = _par_branches(b, _start, _wait)
        base = wid * RW + b * RB
        _build_i3(idx_v, par * RB, i30, i31, i32)
        i3s = (i30, i31, i32)
        gs = [pltpu.async_copy(shm.at[i3s[ci]],
                               mrt_v.at[pl.ds(ci * RB, RB)], sg)
              for ci in range(3)]
        for g in gs:
            g.wait()

        def comb(k, _):
            o = 16 * k
            for ci in range(3):
                po3 = (par * 3 + ci) * RB + o
                oc = ci * RB + o
                rt = _wrap_signed16(
                    rp_v[pl.ds(po3, 16)] - mrt_v[pl.ds(oc, 16)])
                rt_v[pl.ds(oc, 16)] = rt
                ft_v[pl.ds(oc, 16)] = _wrap_frac16(
                    f0_v[pl.ds(po3, 16)] + rt)
            return 0

        lax.fori_loop(0, MB, comb, 0)
        outs = []
        for ci in range(3):
            oc = pl.ds(ci * RB, RB)
            hs = pl.ds(base, RB)
            outs.append(pltpu.async_copy(rt_v.at[oc], rts[ci].at[hs], sout))
            outs.append(pltpu.async_copy(ft_v.at[oc], fts[ci].at[hs], sout))
        for d in outs:
            d.wait()
        return 0

    lax.fori_loop(0, NB, blk, 0)


def _f32(*shape):
    return jax.ShapeDtypeStruct(shape, jnp.float32)


def kernel(t, f0, index, v0, epsilon_v, epsilon_r):
    f0c = [f0[:, i] for i in range(3)]
    v0c = [v0[:, i] for i in range(3)]
    evc = [epsilon_v[:, i] for i in range(3)]
    erc = [epsilon_r[:, i] for i in range(3)]

    sums = pl.kernel(
        _sums_body,
        out_type=(_f32(NC, 3 * S), _f32(NC, 3 * S), _f32(NC, S)),
        mesh=_mesh(),
        compiler_params=_params,
        scratch_types=[
            pltpu.VMEM((2 * RB,), jnp.int32),
            pltpu.VMEM((RB,), jnp.int32),
            pltpu.VMEM((RB,), jnp.int32),
            pltpu.VMEM((RB,), jnp.int32),
            pltpu.VMEM((2 * 3 * RB,), jnp.float32),
            pltpu.VMEM((2 * 3 * RB,), jnp.float32),
            pltpu.VMEM((RB,), jnp.float32),
            pltpu.VMEM((ZL,), jnp.float32),
            pltpu.VMEM_SHARED((3 * S,), jnp.float32),
            pltpu.VMEM_SHARED((3 * S,), jnp.float32),
            pltpu.VMEM_SHARED((S,), jnp.float32),
            pltpu.SemaphoreType.DMA,
            pltpu.SemaphoreType.DMA,
        ],
    )
    psum_v, psum_r, pcnt = sums(index, *evc, *erc)

    means = pl.kernel(
        _means_body,
        out_type=(_f32(3 * S), _f32(3 * S), _f32(S)),
        mesh=_mesh(),
        compiler_params=_params,
        scratch_types=[
            pltpu.VMEM((SWL,), jnp.float32),
            pltpu.VMEM((SWL,), jnp.float32),
            pltpu.VMEM((SWL,), jnp.float32),
            pltpu.VMEM((SWL,), jnp.float32),
            pltpu.VMEM((SWL,), jnp.float32),
        ],
    )
    mean_v, mean_r, cnt_tot = means(psum_v, psum_r, pcnt)

    main = pl.kernel(
        _main_body,
        out_type=tuple([_f32(N)] * 12 + [_f32(NC, 3 * S)]),
        mesh=_mesh(),
        compiler_params=_params,
        scratch_types=[
            pltpu.VMEM((2 * RB,), jnp.int32),
            pltpu.VMEM((RB,), jnp.int32),
            pltpu.VMEM((RB,), jnp.int32),
            pltpu.VMEM((RB,), jnp.int32),
            pltpu.VMEM((2 * RB,), jnp.float32),
            pltpu.VMEM((2 * 3 * RB,), jnp.float32),
            pltpu.VMEM((2 * 3 * RB,), jnp.float32),
            pltpu.VMEM((2 * 3 * RB,), jnp.float32),
            pltpu.VMEM((3 * RB,), jnp.float32),
            pltpu.VMEM((3 * RB,), jnp.float32),
            pltpu.VMEM((3 * RB,), jnp.float32),
            pltpu.VMEM((3 * RB,), jnp.float32),
            pltpu.VMEM((3 * RB,), jnp.float32),
            pltpu.VMEM((3 * RB,), jnp.float32),
            pltpu.VMEM((RB,), jnp.float32),
            pltpu.VMEM((RB,), jnp.float32),
            pltpu.VMEM((RB,), jnp.float32),
            pltpu.VMEM((RB,), jnp.float32),
            pltpu.VMEM((ZL,), jnp.float32),
            pltpu.VMEM_SHARED((3 * S,), jnp.float32),
            pltpu.VMEM_SHARED((3 * S,), jnp.float32),
            pltpu.VMEM_SHARED((3 * S,), jnp.float32),
        ] + [pltpu.SemaphoreType.DMA] * 5,
    )
    outs = main(t, index, *v0c, *evc, *erc, mean_v, mean_r)
    vt = outs[0:3]
    ec = outs[3:6]
    rc = outs[6:9]
    rp = outs[9:12]
    psum_rt = outs[12]

    rt_means = pl.kernel(
        _rt_means_body,
        out_type=_f32(3 * S),
        mesh=_mesh(),
        compiler_params=_params,
        scratch_types=[
            pltpu.VMEM((SWL,), jnp.float32),
            pltpu.VMEM((SWL,), jnp.float32),
            pltpu.VMEM((SWL,), jnp.float32),
            pltpu.VMEM((SWL,), jnp.float32),
        ],
    )
    mean_rt = rt_means(psum_rt, cnt_tot)

    final = pl.kernel(
        _final_body,
        out_type=tuple([_f32(N)] * 6),
        mesh=_mesh(),
        compiler_params=_params,
        scratch_types=[
            pltpu.VMEM((2 * RB,), jnp.int32),
            pltpu.VMEM((RB,), jnp.int32),
            pltpu.VMEM((RB,), jnp.int32),
            pltpu.VMEM((RB,), jnp.int32),
            pltpu.VMEM((2 * 3 * RB,), jnp.float32),
            pltpu.VMEM((2 * 3 * RB,), jnp.float32),
            pltpu.VMEM((3 * RB,), jnp.float32),
            pltpu.VMEM((3 * RB,), jnp.float32),
            pltpu.VMEM((3 * RB,), jnp.float32),
            pltpu.VMEM((ZL,), jnp.float32),
            pltpu.VMEM_SHARED((3 * S,), jnp.float32),
        ] + [pltpu.SemaphoreType.DMA] * 4,
    )
    fouts = final(*f0c, *rp, index, mean_rt)
    rt = fouts[0:3]
    ft = fouts[3:6]

    stack = lambda cols: jnp.stack(cols, axis=1)
    return (stack(ft), stack(vt), stack(ec), stack(rc), stack(rt))


# speedup vs baseline: 15.3753x; 1.0593x over previous
import functools

import jax
import jax.numpy as jnp
from jax import lax
from jax.experimental import pallas as pl
from jax.experimental.pallas import tpu as pltpu
from jax.experimental.pallas import tpu_sc as plsc

N = 1600000
S = 32768
EPS = 1e-05
T_SCALE = 2.0

NC = 2
NS = 16
NW = NC * NS
RW = N // NW
RB = 2000
NB = RW // RB
MB = RB // 16
SWL = S // NW
ZL = 3 * S // NS
CL = S // NS

_params = pltpu.CompilerParams(needs_layout_passes=False)

_mesh = functools.partial(
    plsc.VectorSubcoreMesh, core_axis_name="c", subcore_axis_name="s",
    num_cores=NC, num_subcores=NS)


def _wid():
    c = lax.axis_index("c")
    s = lax.axis_index("s")
    return s * NC + c, c, s


def _zero_fill(ref, n):
    z = jnp.zeros((16,), jnp.float32)

    def body(k, _):
        ref[pl.ds(16 * k, 16)] = z
        return 0

    lax.fori_loop(0, n // 16, body, 0)


def _build_i3(idx_v, ioff, i30, i31, i32):
    def body(k, _):
        seg = idx_v[pl.ds(ioff + 16 * k, 16)]
        i30[pl.ds(16 * k, 16)] = seg
        i31[pl.ds(16 * k, 16)] = seg + S
        i32[pl.ds(16 * k, 16)] = seg + 2 * S
        return 0

    lax.fori_loop(0, MB, body, 0)


def _sqrt16(x):
    y = plsc.bitcast(
        jnp.int32(0x5F3759DF) - (plsc.bitcast(x, jnp.int32) >> 1), jnp.float32)
    half = x * 0.5
    for _ in range(3):
        y = y * (1.5 - half * y * y)
    return x * y


def _floor16(x):
    t = lax.convert_element_type(
        lax.convert_element_type(x, jnp.int32), jnp.float32)
    return jnp.where(t > x, t - 1.0, t)


def _wrap_signed16(x):
    y = x + 0.5
    return (y - _floor16(y)) - 0.5


def _wrap_frac16(x):
    return x - _floor16(x)


def _par_branches(b, start, wait):
    par = lax.rem(b, 2)
    nb_ok = b + 1 < NB

    @pl.when(jnp.logical_and(nb_ok, par == 0))
    def _():
        start(b + 1, 1)

    @pl.when(jnp.logical_and(nb_ok, par == 1))
    def _():
        start(b + 1, 0)

    @pl.when(par == 0)
    def _():
        wait(b, 0)

    @pl.when(par == 1)
    def _():
        wait(b, 1)

    return par


def _sums_body(index, ev0, ev1, ev2, er0, er1, er2, psum_v, psum_r, pcnt,
               idx_v, i30, i31, i32, ev_v, er_v, ones_v, zb_v,
               acc_v, acc_r, acc_c, sin0, sin1):
    wid, c, s = _wid()
    sems = (sin0, sin1)
    evs = (ev0, ev1, ev2)
    ers = (er0, er1, er2)

    one = jnp.full((16,), 1.0, jnp.float32)

    def ones_body(k, _):
        ones_v[pl.ds(16 * k, 16)] = one
        return 0

    lax.fori_loop(0, RB // 16, ones_body, 0)

    _zero_fill(zb_v, ZL)
    pltpu.sync_copy(zb_v, acc_v.at[pl.ds(s * ZL, ZL)])
    pltpu.sync_copy(zb_v, acc_r.at[pl.ds(s * ZL, ZL)])
    pltpu.sync_copy(zb_v.at[pl.ds(0, CL)], acc_c.at[pl.ds(s * CL, CL)])
    plsc.subcore_barrier()

    def _dmas(b, par):
        base = wid * RW + b * RB
        sem = sems[par]
        ds = [pltpu.make_async_copy(index.at[pl.ds(base, RB)],
                                    idx_v.at[pl.ds(par * RB, RB)], sem)]
        for ci in range(3):
            ds.append(pltpu.make_async_copy(
                evs[ci].at[pl.ds(base, RB)],
                ev_v.at[pl.ds((par * 3 + ci) * RB, RB)], sem))
            ds.append(pltpu.make_async_copy(
                ers[ci].at[pl.ds(base, RB)],
                er_v.at[pl.ds((par * 3 + ci) * RB, RB)], sem))
        return ds

    def _start(b, par):
        for d in _dmas(b, par):
            d.start()

    def _wait(b, par):
        for d in _dmas(b, par):
            d.wait()

    _start(0, 0)

    def blk(b, _):
        par = _par_branches(b, _start, _wait)
        _build_i3(idx_v, par * RB, i30, i31, i32)
        i3s = (i30, i31, i32)
        for ci in range(3):
            pltpu.sync_copy(ev_v.at[pl.ds((par * 3 + ci) * RB, RB)],
                            acc_v.at[i3s[ci]], add=True)
            pltpu.sync_copy(er_v.at[pl.ds((par * 3 + ci) * RB, RB)],
                            acc_r.at[i3s[ci]], add=True)
        pltpu.sync_copy(ones_v, acc_c.at[i30], add=True)
        return 0

    lax.fori_loop(0, NB, blk, 0)
    plsc.subcore_barrier()

    pltpu.sync_copy(acc_v.at[pl.ds(s * ZL, ZL)], zb_v)
    pltpu.sync_copy(zb_v, psum_v.at[c, pl.ds(s * ZL, ZL)])
    pltpu.sync_copy(acc_r.at[pl.ds(s * ZL, ZL)], zb_v)
    pltpu.sync_copy(zb_v, psum_r.at[c, pl.ds(s * ZL, ZL)])
    pltpu.sync_copy(acc_c.at[pl.ds(s * CL, CL)], zb_v.at[pl.ds(0, CL)])
    pltpu.sync_copy(zb_v.at[pl.ds(0, CL)], pcnt.at[c, pl.ds(s * CL, CL)])


def _stage_mean(src, rc_v, a_v, b_v, s, dst):
    for ci in range(3):
        o = ci * S + s * CL
        pltpu.sync_copy(src.at[0, pl.ds(o, CL)], a_v)
        pltpu.sync_copy(src.at[1, pl.ds(o, CL)], b_v)

        def mbody(k, _):
            a_v[pl.ds(16 * k, 16)] = (
                a_v[pl.ds(16 * k, 16)] + b_v[pl.ds(16 * k, 16)]
            ) * rc_v[pl.ds(16 * k, 16)]
            return 0

        lax.fori_loop(0, CL // 16, mbody, 0)
        pltpu.sync_copy(a_v, dst.at[pl.ds(o, CL)])


def _recip_counts(pcnt, rc_v, b_v, s):
    pltpu.sync_copy(pcnt.at[0, pl.ds(s * CL, CL)], rc_v)
    pltpu.sync_copy(pcnt.at[1, pl.ds(s * CL, CL)], b_v)

    def cbody(k, _):
        rc_v[pl.ds(16 * k, 16)] = 1.0 / jnp.maximum(
            rc_v[pl.ds(16 * k, 16)] + b_v[pl.ds(16 * k, 16)], 1.0)
        return 0

    lax.fori_loop(0, CL // 16, cbody, 0)


def _main_body(t, index, v00, v01, v02, ev0, ev1, ev2, er0, er1, er2,
               psum_v, psum_r, pcnt,
               vt0, vt1, vt2, ec0, ec1, ec2, rc0, rc1, rc2,
               rp0, rp1, rp2, psum_rt,
               idx_v, i30, i31, i32, t_v, v0_v, ev_v, er_v, mv_v, mr_v,
               vt_v, evc_v, erc_v, rp_v, al_v, sg_v, co_v, sr_v, zb_v,
               ca_v, cb_v, crc_v,
               acc_rt, shv, shr, sin0, sin1, sgv, sgr, sout):
    wid, c, s = _wid()
    sems = (sin0, sin1)
    v0s = (v00, v01, v02)
    evs = (ev0, ev1, ev2)
    ers = (er0, er1, er2)
    vts = (vt0, vt1, vt2)
    ecs = (ec0, ec1, ec2)
    rcs = (rc0, rc1, rc2)
    rps = (rp0, rp1, rp2)

    _zero_fill(zb_v, ZL)
    pltpu.sync_copy(zb_v, acc_rt.at[pl.ds(s * ZL, ZL)])
    _recip_counts(pcnt, crc_v, cb_v, s)
    _stage_mean(psum_v, crc_v, ca_v, cb_v, s, shv)
    _stage_mean(psum_r, crc_v, ca_v, cb_v, s, shr)
    plsc.subcore_barrier()

    def _dmas(b, par):
        base = wid * RW + b * RB
        sem = sems[par]
        ds = [
            pltpu.make_async_copy(index.at[pl.ds(base, RB)],
                                  idx_v.at[pl.ds(par * RB, RB)], sem),
            pltpu.make_async_copy(t.at[pl.ds(base, RB)],
                                  t_v.at[pl.ds(par * RB, RB)], sem),
        ]
        for ci in range(3):
            po = (par * 3 + ci) * RB
            ds.append(pltpu.make_async_copy(
                v0s[ci].at[pl.ds(base, RB)], v0_v.at[pl.ds(po, RB)], sem))
            ds.append(pltpu.make_async_copy(
                evs[ci].at[pl.ds(base, RB)], ev_v.at[pl.ds(po, RB)], sem))
            ds.append(pltpu.make_async_copy(
                ers[ci].at[pl.ds(base, RB)], er_v.at[pl.ds(po, RB)], sem))
        return ds

    def _start(b, par):
        for d in _dmas(b, par):
            d.start()

    def _wait(b, par):
        for d in _dmas(b, par):
            d.wait()

    _start(0, 0)

    def blk(b, _):
        par = _par_branches(b, _start, _wait)
        po = par * RB
        base = wid * RW + b * RB
        _build_i3(idx_v, po, i30, i31, i32)
        i3s = (i30, i31, i32)
        gs = []
        for ci in range(3):
            gs.append(pltpu.async_copy(
                shv.at[i3s[ci]], mv_v.at[pl.ds(ci * RB, RB)], sgv))
            gs.append(pltpu.async_copy(
                shr.at[i3s[ci]], mr_v.at[pl.ds(ci * RB, RB)], sgr))

        def coef(k, _):
            ts = T_SCALE * t_v[pl.ds(po + 16 * k, 16)]
            e = jnp.exp(-ts)
            al_v[pl.ds(16 * k, 16)] = e
            sg_v[pl.ds(16 * k, 16)] = _sqrt16(jnp.maximum(1.0 - e * e, EPS))
            co_v[pl.ds(16 * k, 16)] = (1.0 - e) / (1.0 + e)
            sr_v[pl.ds(16 * k, 16)] = _sqrt16(
                jnp.maximum(2.0 * ts + 8.0 * e / (1.0 + e) - 4.0, EPS))
            return 0

        lax.fori_loop(0, MB, coef, 0)
        for g in gs:
            g.wait()

        def comb(k, _):
            o = 16 * k
            al = al_v[pl.ds(o, 16)]
            sg = sg_v[pl.ds(o, 16)]
            co = co_v[pl.ds(o, 16)]
            sr = sr_v[pl.ds(o, 16)]
            for ci in range(3):
                po3 = (par * 3 + ci) * RB + o
                oc = ci * RB + o
                ec = ev_v[pl.ds(po3, 16)] - mv_v[pl.ds(oc, 16)]
                evc_v[pl.ds(oc, 16)] = ec
                rc = er_v[pl.ds(po3, 16)] - mr_v[pl.ds(oc, 16)]
                erc_v[pl.ds(oc, 16)] = rc
                v0x = v0_v[pl.ds(po3, 16)]
                vt = al * v0x + sg * ec
                vt_v[pl.ds(oc, 16)] = vt
                rp_v[pl.ds(oc, 16)] = _wrap_signed16(
                    co * (vt + v0x) + sr * rc)
            return 0

        lax.fori_loop(0, MB, comb, 0)

        outs = []
        for ci in range(3):
            oc = pl.ds(ci * RB, RB)
            hs = pl.ds(base, RB)
            outs.append(pltpu.async_copy(vt_v.at[oc], vts[ci].at[hs], sout))
            outs.append(pltpu.async_copy(evc_v.at[oc], ecs[ci].at[hs], sout))
            outs.append(pltpu.async_copy(erc_v.at[oc], rcs[ci].at[hs], sout))
            outs.append(pltpu.async_copy(rp_v.at[oc], rps[ci].at[hs], sout))
            pltpu.sync_copy(rp_v.at[oc], acc_rt.at[i3s[ci]], add=True)
        for d in outs:
            d.wait()
        return 0

    lax.fori_loop(0, NB, blk, 0)
    plsc.subcore_barrier()

    pltpu.sync_copy(acc_rt.at[pl.ds(s * ZL, ZL)], zb_v)
    pltpu.sync_copy(zb_v, psum_rt.at[c, pl.ds(s * ZL, ZL)])


def _final_body(f00, f01, f02, rp0, rp1, rp2, index, psum_rt, pcnt,
                rt0, rt1, rt2, ft0, ft1, ft2,
                idx_v, i30, i31, i32, f0_v, rp_v, mrt_v, rt_v, ft_v,
                ca_v, cb_v, crc_v,
                shm, sin0, sin1, sg, sout):
    wid, c, s = _wid()
    sems = (sin0, sin1)
    f0s = (f00, f01, f02)
    rps = (rp0, rp1, rp2)
    rts = (rt0, rt1, rt2)
    fts = (ft0, ft1, ft2)

    _recip_counts(pcnt, crc_v, cb_v, s)
    _stage_mean(psum_rt, crc_v, ca_v, cb_v, s, shm)
    plsc.subcore_barrier()

    def _dmas(b, par):
        base = wid * RW + b * RB
        sem = sems[par]
        ds = [pltpu.make_async_copy(index.at[pl.ds(base, RB)],
                                    idx_v.at[pl.ds(par * RB, RB)], sem)]
        for ci in range(3):
            po = (par * 3 + ci) * RB
            ds.append(pltpu.make_async_copy(
                f0s[ci].at[pl.ds(base, RB)], f0_v.at[pl.ds(po, RB)], sem))
            ds.append(pltpu.make_async_copy(
                rps[ci].at[pl.ds(base, RB)], rp_v.at[pl.ds(po, RB)], sem))
        return ds

    def _start(b, par):
        for d in _dmas(b, par):
            d.start()

    def _wait(b, par):
        for d in _dmas(b, par):
            d.wait()

    _start(0, 0)

    def blk(b, _):
        par = _par_branches(b, _start, _wait)
        base = wid * RW + b * RB
        _build_i3(idx_v, par * RB, i30, i31, i32)
        i3s = (i30, i31, i32)
        gs = [pltpu.async_copy(shm.at[i3s[ci]],
                               mrt_v.at[pl.ds(ci * RB, RB)], sg)
              for ci in range(3)]
        for g in gs:
            g.wait()

        def comb(k, _):
            o = 16 * k
            for ci in range(3):
                po3 = (par * 3 + ci) * RB + o
                oc = ci * RB + o
                rt = _wrap_signed16(
                    rp_v[pl.ds(po3, 16)] - mrt_v[pl.ds(oc, 16)])
                rt_v[pl.ds(oc, 16)] = rt
                ft_v[pl.ds(oc, 16)] = _wrap_frac16(
                    f0_v[pl.ds(po3, 16)] + rt)
            return 0

        lax.fori_loop(0, MB, comb, 0)
        outs = []
        for ci in range(3):
            oc = pl.ds(ci * RB, RB)
            hs = pl.ds(base, RB)
            outs.append(pltpu.async_copy(rt_v.at[oc], rts[ci].at[hs], sout))
            outs.append(pltpu.async_copy(ft_v.at[oc], fts[ci].at[hs], sout))
        for d in outs:
            d.wait()
        return 0

    lax.fori_loop(0, NB, blk, 0)


def _f32(*shape):
    return jax.ShapeDtypeStruct(shape, jnp.float32)


def kernel(t, f0, index, v0, epsilon_v, epsilon_r):
    f0c = [f0[:, i] for i in range(3)]
    v0c = [v0[:, i] for i in range(3)]
    evc = [epsilon_v[:, i] for i in range(3)]
    erc = [epsilon_r[:, i] for i in range(3)]

    sums = pl.kernel(
        _sums_body,
        out_type=(_f32(NC, 3 * S), _f32(NC, 3 * S), _f32(NC, S)),
        mesh=_mesh(),
        compiler_params=_params,
        scratch_types=[
            pltpu.VMEM((2 * RB,), jnp.int32),
            pltpu.VMEM((RB,), jnp.int32),
            pltpu.VMEM((RB,), jnp.int32),
            pltpu.VMEM((RB,), jnp.int32),
            pltpu.VMEM((2 * 3 * RB,), jnp.float32),
            pltpu.VMEM((2 * 3 * RB,), jnp.float32),
            pltpu.VMEM((RB,), jnp.float32),
            pltpu.VMEM((ZL,), jnp.float32),
            pltpu.VMEM_SHARED((3 * S,), jnp.float32),
            pltpu.VMEM_SHARED((3 * S,), jnp.float32),
            pltpu.VMEM_SHARED((S,), jnp.float32),
            pltpu.SemaphoreType.DMA,
            pltpu.SemaphoreType.DMA,
        ],
    )
    psum_v, psum_r, pcnt = sums(index, *evc, *erc)

    main = pl.kernel(
        _main_body,
        out_type=tuple([_f32(N)] * 12 + [_f32(NC, 3 * S)]),
        mesh=_mesh(),
        compiler_params=_params,
        scratch_types=[
            pltpu.VMEM((2 * RB,), jnp.int32),
            pltpu.VMEM((RB,), jnp.int32),
            pltpu.VMEM((RB,), jnp.int32),
            pltpu.VMEM((RB,), jnp.int32),
            pltpu.VMEM((2 * RB,), jnp.float32),
            pltpu.VMEM((2 * 3 * RB,), jnp.float32),
            pltpu.VMEM((2 * 3 * RB,), jnp.float32),
            pltpu.VMEM((2 * 3 * RB,), jnp.float32),
            pltpu.VMEM((3 * RB,), jnp.float32),
            pltpu.VMEM((3 * RB,), jnp.float32),
            pltpu.VMEM((3 * RB,), jnp.float32),
            pltpu.VMEM((3 * RB,), jnp.float32),
            pltpu.VMEM((3 * RB,), jnp.float32),
            pltpu.VMEM((3 * RB,), jnp.float32),
            pltpu.VMEM((RB,), jnp.float32),
            pltpu.VMEM((RB,), jnp.float32),
            pltpu.VMEM((RB,), jnp.float32),
            pltpu.VMEM((RB,), jnp.float32),
            pltpu.VMEM((ZL,), jnp.float32),
            pltpu.VMEM((CL,), jnp.float32),
            pltpu.VMEM((CL,), jnp.float32),
            pltpu.VMEM((CL,), jnp.float32),
            pltpu.VMEM_SHARED((3 * S,), jnp.float32),
            pltpu.VMEM_SHARED((3 * S,), jnp.float32),
            pltpu.VMEM_SHARED((3 * S,), jnp.float32),
        ] + [pltpu.SemaphoreType.DMA] * 5,
    )
    outs = main(t, index, *v0c, *evc, *erc, psum_v, psum_r, pcnt)
    vt = outs[0:3]
    ec = outs[3:6]
    rc = outs[6:9]
    rp = outs[9:12]
    psum_rt = outs[12]

    final = pl.kernel(
        _final_body,
        out_type=tuple([_f32(N)] * 6),
        mesh=_mesh(),
        compiler_params=_params,
        scratch_types=[
            pltpu.VMEM((2 * RB,), jnp.int32),
            pltpu.VMEM((RB,), jnp.int32),
            pltpu.VMEM((RB,), jnp.int32),
            pltpu.VMEM((RB,), jnp.int32),
            pltpu.VMEM((2 * 3 * RB,), jnp.float32),
            pltpu.VMEM((2 * 3 * RB,), jnp.float32),
            pltpu.VMEM((3 * RB,), jnp.float32),
            pltpu.VMEM((3 * RB,), jnp.float32),
            pltpu.VMEM((3 * RB,), jnp.float32),
            pltpu.VMEM((CL,), jnp.float32),
            pltpu.VMEM((CL,), jnp.float32),
            pltpu.VMEM((CL,), jnp.float32),
            pltpu.VMEM_SHARED((3 * S,), jnp.float32),
        ] + [pltpu.SemaphoreType.DMA] * 4,
    )
    fouts = final(*f0c, *rp, index, psum_rt, pcnt)
    rt = fouts[0:3]
    ft = fouts[3:6]

    stack = lambda cols: jnp.stack(cols, axis=1)
    return (stack(ft), stack(vt), stack(ec), stack(rc), stack(rt))


# speedup vs baseline: 16.9107x; 1.0999x over previous
import functools

import jax
import jax.numpy as jnp
from jax import lax
from jax.experimental import pallas as pl
from jax.experimental.pallas import tpu as pltpu
from jax.experimental.pallas import tpu_sc as plsc

N = 1600000
S = 32768
EPS = 1e-05
T_SCALE = 2.0

NC = 2
NS = 16
NW = NC * NS
RW = N // NW
RB = 2000
NB = RW // RB
MB = RB // 16
SWL = S // NW
ZL = 3 * S // NS
CL = S // NS
GCAP = 1024

_params = pltpu.CompilerParams(needs_layout_passes=False)

_mesh = functools.partial(
    plsc.VectorSubcoreMesh, core_axis_name="c", subcore_axis_name="s",
    num_cores=NC, num_subcores=NS)


def _wid():
    c = lax.axis_index("c")
    s = lax.axis_index("s")
    return s * NC + c, c, s


def _zero_fill(ref, n):
    z = jnp.zeros((16,), jnp.float32)

    def body(k, _):
        ref[pl.ds(16 * k, 16)] = z
        return 0

    lax.fori_loop(0, n // 16, body, 0)


def _build_i3(idx_v, ioff, i30, i31, i32):
    def body(k, _):
        seg = idx_v[pl.ds(ioff + 16 * k, 16)]
        i30[pl.ds(16 * k, 16)] = seg
        i31[pl.ds(16 * k, 16)] = seg + S
        i32[pl.ds(16 * k, 16)] = seg + 2 * S
        return 0

    lax.fori_loop(0, MB, body, 0)


def _sqrt16(x):
    y = plsc.bitcast(
        jnp.int32(0x5F3759DF) - (plsc.bitcast(x, jnp.int32) >> 1), jnp.float32)
    half = x * 0.5
    for _ in range(3):
        y = y * (1.5 - half * y * y)
    return x * y


def _floor16(x):
    t = lax.convert_element_type(
        lax.convert_element_type(x, jnp.int32), jnp.float32)
    return jnp.where(t > x, t - 1.0, t)


def _wrap_signed16(x):
    y = x + 0.5
    return (y - _floor16(y)) - 0.5


def _wrap_frac16(x):
    return x - _floor16(x)


def _par_branches(b, start, wait):
    par = lax.rem(b, 2)
    nb_ok = b + 1 < NB

    @pl.when(jnp.logical_and(nb_ok, par == 0))
    def _():
        start(b + 1, 1)

    @pl.when(jnp.logical_and(nb_ok, par == 1))
    def _():
        start(b + 1, 0)

    @pl.when(par == 0)
    def _():
        wait(b, 0)

    @pl.when(par == 1)
    def _():
        wait(b, 1)

    return par


def _sums_body(index, ev0, ev1, ev2, er0, er1, er2, psum_v, psum_r, pcnt,
               idx_v, i30, i31, i32, ev_v, er_v, ones_v, zb_v,
               acc_v, acc_r, acc_c, sin0, sin1):
    wid, c, s = _wid()
    sems = (sin0, sin1)
    evs = (ev0, ev1, ev2)
    ers = (er0, er1, er2)

    one = jnp.full((16,), 1.0, jnp.float32)

    def ones_body(k, _):
        ones_v[pl.ds(16 * k, 16)] = one
        return 0

    lax.fori_loop(0, RB // 16, ones_body, 0)

    _zero_fill(zb_v, ZL)
    pltpu.sync_copy(zb_v, acc_v.at[pl.ds(s * ZL, ZL)])
    pltpu.sync_copy(zb_v, acc_r.at[pl.ds(s * ZL, ZL)])
    pltpu.sync_copy(zb_v.at[pl.ds(0, CL)], acc_c.at[pl.ds(s * CL, CL)])
    plsc.subcore_barrier()

    def _dmas(b, par):
        base = wid * RW + b * RB
        sem = sems[par]
        ds = [pltpu.make_async_copy(index.at[pl.ds(base, RB)],
                                    idx_v.at[pl.ds(par * RB, RB)], sem)]
        for ci in range(3):
            ds.append(pltpu.make_async_copy(
                evs[ci].at[pl.ds(base, RB)],
                ev_v.at[pl.ds((par * 3 + ci) * RB, RB)], sem))
            ds.append(pltpu.make_async_copy(
                ers[ci].at[pl.ds(base, RB)],
                er_v.at[pl.ds((par * 3 + ci) * RB, RB)], sem))
        return ds

    def _start(b, par):
        for d in _dmas(b, par):
            d.start()

    def _wait(b, par):
        for d in _dmas(b, par):
            d.wait()

    _start(0, 0)

    def blk(b, _):
        par = _par_branches(b, _start, _wait)
        _build_i3(idx_v, par * RB, i30, i31, i32)
        i3s = (i30, i31, i32)
        for ci in range(3):
            pltpu.sync_copy(ev_v.at[pl.ds((par * 3 + ci) * RB, RB)],
                            acc_v.at[i3s[ci]], add=True)
            pltpu.sync_copy(er_v.at[pl.ds((par * 3 + ci) * RB, RB)],
                            acc_r.at[i3s[ci]], add=True)
        pltpu.sync_copy(ones_v, acc_c.at[i30], add=True)
        return 0

    lax.fori_loop(0, NB, blk, 0)
    plsc.subcore_barrier()

    pltpu.sync_copy(acc_v.at[pl.ds(s * ZL, ZL)], zb_v)
    pltpu.sync_copy(zb_v, psum_v.at[c, pl.ds(s * ZL, ZL)])
    pltpu.sync_copy(acc_r.at[pl.ds(s * ZL, ZL)], zb_v)
    pltpu.sync_copy(zb_v, psum_r.at[c, pl.ds(s * ZL, ZL)])
    pltpu.sync_copy(acc_c.at[pl.ds(s * CL, CL)], zb_v.at[pl.ds(0, CL)])
    pltpu.sync_copy(zb_v.at[pl.ds(0, CL)], pcnt.at[c, pl.ds(s * CL, CL)])


def _stage_mean(src, rc_v, a_v, b_v, s, dst):
    for ci in range(3):
        o = ci * S + s * CL
        pltpu.sync_copy(src.at[0, pl.ds(o, CL)], a_v)
        pltpu.sync_copy(src.at[1, pl.ds(o, CL)], b_v)

        def mbody(k, _):
            a_v[pl.ds(16 * k, 16)] = (
                a_v[pl.ds(16 * k, 16)] + b_v[pl.ds(16 * k, 16)]
            ) * rc_v[pl.ds(16 * k, 16)]
            return 0

        lax.fori_loop(0, CL // 16, mbody, 0)
        pltpu.sync_copy(a_v, dst.at[pl.ds(o, CL)])


def _recip_counts(pcnt, rc_v, b_v, s):
    pltpu.sync_copy(pcnt.at[0, pl.ds(s * CL, CL)], rc_v)
    pltpu.sync_copy(pcnt.at[1, pl.ds(s * CL, CL)], b_v)

    def cbody(k, _):
        rc_v[pl.ds(16 * k, 16)] = 1.0 / jnp.maximum(
            rc_v[pl.ds(16 * k, 16)] + b_v[pl.ds(16 * k, 16)], 1.0)
        return 0

    lax.fori_loop(0, CL // 16, cbody, 0)


def _main_body(t, index, v00, v01, v02, ev0, ev1, ev2, er0, er1, er2,
               psum_v, psum_r, pcnt,
               vt0, vt1, vt2, ec0, ec1, ec2, rc0, rc1, rc2,
               rp0, rp1, rp2, psum_rt,
               idx_v, i30, i31, i32, t_v, v0_v, ev_v, er_v, mv_v, mr_v,
               vt_v, evc_v, erc_v, rp_v, al_v, sg_v, co_v, sr_v, zb_v,
               ca_v, cb_v, crc_v,
               acc_rt, shv, shr, sin0, sin1, sgv, sgr, sout):
    wid, c, s = _wid()
    sems = (sin0, sin1)
    v0s = (v00, v01, v02)
    evs = (ev0, ev1, ev2)
    ers = (er0, er1, er2)
    vts = (vt0, vt1, vt2)
    ecs = (ec0, ec1, ec2)
    rcs = (rc0, rc1, rc2)
    rps = (rp0, rp1, rp2)

    _zero_fill(zb_v, ZL)
    pltpu.sync_copy(zb_v, acc_rt.at[pl.ds(s * ZL, ZL)])
    _recip_counts(pcnt, crc_v, cb_v, s)
    _stage_mean(psum_v, crc_v, ca_v, cb_v, s, shv)
    _stage_mean(psum_r, crc_v, ca_v, cb_v, s, shr)
    plsc.subcore_barrier()

    def _dmas(b, par):
        base = wid * RW + b * RB
        sem = sems[par]
        ds = [
            pltpu.make_async_copy(index.at[pl.ds(base, RB)],
                                  idx_v.at[pl.ds(par * RB, RB)], sem),
            pltpu.make_async_copy(t.at[pl.ds(base, RB)],
                                  t_v.at[pl.ds(par * RB, RB)], sem),
        ]
        for ci in range(3):
            po = (par * 3 + ci) * RB
            ds.append(pltpu.make_async_copy(
                v0s[ci].at[pl.ds(base, RB)], v0_v.at[pl.ds(po, RB)], sem))
            ds.append(pltpu.make_async_copy(
                evs[ci].at[pl.ds(base, RB)], ev_v.at[pl.ds(po, RB)], sem))
            ds.append(pltpu.make_async_copy(
                ers[ci].at[pl.ds(base, RB)], er_v.at[pl.ds(po, RB)], sem))
        return ds

    def _start(b, par):
        for d in _dmas(b, par):
            d.start()

    def _wait(b, par):
        for d in _dmas(b, par):
            d.wait()

    _start(0, 0)

    i16 = lax.iota(jnp.int32, 16)

    def blk(b, _):
        par = _par_branches(b, _start, _wait)
        po = par * RB
        base = wid * RW + b * RB
        _build_i3(idx_v, po, i30, i31, i32)
        i3s = (i30, i31, i32)
        s0 = jnp.min(idx_v[pl.ds(po, 16)])
        s1 = jnp.max(idx_v[pl.ds(po + RB - 16, 16)])
        s0a = pl.multiple_of(jnp.minimum(s0 - lax.rem(s0, 8), S - GCAP), 8)
        narrow = (s1 - s0a) < GCAP

        @pl.when(narrow)
        def _():
            for ci in range(3):
                pltpu.sync_copy(shv.at[pl.ds(ci * S + s0a, GCAP)],
                                mv_v.at[pl.ds(ci * RB, GCAP)])
                pltpu.sync_copy(shr.at[pl.ds(ci * S + s0a, GCAP)],
                                mr_v.at[pl.ds(ci * RB, GCAP)])

        @pl.when(jnp.logical_not(narrow))
        def _():
            for ci in range(3):
                pltpu.sync_copy(shv.at[i3s[ci]],
                                mv_v.at[pl.ds(ci * RB, RB)])
                pltpu.sync_copy(shr.at[i3s[ci]],
                                mr_v.at[pl.ds(ci * RB, RB)])

        def coef(k, _):
            ts = T_SCALE * t_v[pl.ds(po + 16 * k, 16)]
            e = jnp.exp(-ts)
            al_v[pl.ds(16 * k, 16)] = e
            sg_v[pl.ds(16 * k, 16)] = _sqrt16(jnp.maximum(1.0 - e * e, EPS))
            co_v[pl.ds(16 * k, 16)] = (1.0 - e) / (1.0 + e)
            sr_v[pl.ds(16 * k, 16)] = _sqrt16(
                jnp.maximum(2.0 * ts + 8.0 * e / (1.0 + e) - 4.0, EPS))
            return 0

        lax.fori_loop(0, MB, coef, 0)

        def comb(k, _):
            o = 16 * k
            al = al_v[pl.ds(o, 16)]
            sg = sg_v[pl.ds(o, 16)]
            co = co_v[pl.ds(o, 16)]
            sr = sr_v[pl.ds(o, 16)]
            idx16 = idx_v[pl.ds(po + o, 16)]
            d16 = jnp.where(narrow, idx16 - s0a, o + i16)
            for ci in range(3):
                po3 = (par * 3 + ci) * RB + o
                mv = plsc.load_gather(mv_v, [ci * RB + d16])
                mr = plsc.load_gather(mr_v, [ci * RB + d16])
                oc = ci * RB + o
                ec = ev_v[pl.ds(po3, 16)] - mv
                evc_v[pl.ds(oc, 16)] = ec
                rc = er_v[pl.ds(po3, 16)] - mr
                erc_v[pl.ds(oc, 16)] = rc
                v0x = v0_v[pl.ds(po3, 16)]
                vt = al * v0x + sg * ec
                vt_v[pl.ds(oc, 16)] = vt
                rp_v[pl.ds(oc, 16)] = _wrap_signed16(
                    co * (vt + v0x) + sr * rc)
            return 0

        lax.fori_loop(0, MB, comb, 0)

        outs = []
        for ci in range(3):
            oc = pl.ds(ci * RB, RB)
            hs = pl.ds(base, RB)
            outs.append(pltpu.async_copy(vt_v.at[oc], vts[ci].at[hs], sout))
            outs.append(pltpu.async_copy(evc_v.at[oc], ecs[ci].at[hs], sout))
            outs.append(pltpu.async_copy(erc_v.at[oc], rcs[ci].at[hs], sout))
            outs.append(pltpu.async_copy(rp_v.at[oc], rps[ci].at[hs], sout))
            pltpu.sync_copy(rp_v.at[oc], acc_rt.at[i3s[ci]], add=True)
        for d in outs:
            d.wait()
        return 0

    lax.fori_loop(0, NB, blk, 0)
    plsc.subcore_barrier()

    pltpu.sync_copy(acc_rt.at[pl.ds(s * ZL, ZL)], zb_v)
    pltpu.sync_copy(zb_v, psum_rt.at[c, pl.ds(s * ZL, ZL)])


def _final_body(f00, f01, f02, rp0, rp1, rp2, index, psum_rt, pcnt,
                rt0, rt1, rt2, ft0, ft1, ft2,
                idx_v, i30, i31, i32, f0_v, rp_v, mrt_v, rt_v, ft_v,
                ca_v, cb_v, crc_v,
                shm, sin0, sin1, sg, sout):
    wid, c, s = _wid()
    sems = (sin0, sin1)
    f0s = (f00, f01, f02)
    rps = (rp0, rp1, rp2)
    rts = (rt0, rt1, rt2)
    fts = (ft0, ft1, ft2)

    _recip_counts(pcnt, crc_v, cb_v, s)
    _stage_mean(psum_rt, crc_v, ca_v, cb_v, s, shm)
    plsc.subcore_barrier()

    def _dmas(b, par):
        base = wid * RW + b * RB
        sem = sems[par]
        ds = [pltpu.make_async_copy(index.at[pl.ds(base, RB)],
                                    idx_v.at[pl.ds(par * RB, RB)], sem)]
        for ci in range(3):
            po = (par * 3 + ci) * RB
            ds.append(pltpu.make_async_copy(
                f0s[ci].at[pl.ds(base, RB)], f0_v.at[pl.ds(po, RB)], sem))
            ds.append(pltpu.make_async_copy(
                rps[ci].at[pl.ds(base, RB)], rp_v.at[pl.ds(po, RB)], sem))
        return ds

    def _start(b, par):
        for d in _dmas(b, par):
            d.start()

    def _wait(b, par):
        for d in _dmas(b, par):
            d.wait()

    _start(0, 0)

    i16 = lax.iota(jnp.int32, 16)

    def blk(b, _):
        par = _par_branches(b, _start, _wait)
        po = par * RB
        base = wid * RW + b * RB
        _build_i3(idx_v, po, i30, i31, i32)
        i3s = (i30, i31, i32)
        s0 = jnp.min(idx_v[pl.ds(po, 16)])
        s1 = jnp.max(idx_v[pl.ds(po + RB - 16, 16)])
        s0a = pl.multiple_of(jnp.minimum(s0 - lax.rem(s0, 8), S - GCAP), 8)
        narrow = (s1 - s0a) < GCAP

        @pl.when(narrow)
        def _():
            for ci in range(3):
                pltpu.sync_copy(shm.at[pl.ds(ci * S + s0a, GCAP)],
                                mrt_v.at[pl.ds(ci * RB, GCAP)])

        @pl.when(jnp.logical_not(narrow))
        def _():
            for ci in range(3):
                pltpu.sync_copy(shm.at[i3s[ci]],
                                mrt_v.at[pl.ds(ci * RB, RB)])

        def comb(k, _):
            o = 16 * k
            idx16 = idx_v[pl.ds(po + o, 16)]
            d16 = jnp.where(narrow, idx16 - s0a, o + i16)
            for ci in range(3):
                po3 = (par * 3 + ci) * RB + o
                mrt = plsc.load_gather(mrt_v, [ci * RB + d16])
                oc = ci * RB + o
                rt = _wrap_signed16(rp_v[pl.ds(po3, 16)] - mrt)
                rt_v[pl.ds(oc, 16)] = rt
                ft_v[pl.ds(oc, 16)] = _wrap_frac16(
                    f0_v[pl.ds(po3, 16)] + rt)
            return 0

        lax.fori_loop(0, MB, comb, 0)
        outs = []
        for ci in range(3):
            oc = pl.ds(ci * RB, RB)
            hs = pl.ds(base, RB)
            outs.append(pltpu.async_copy(rt_v.at[oc], rts[ci].at[hs], sout))
            outs.append(pltpu.async_copy(ft_v.at[oc], fts[ci].at[hs], sout))
        for d in outs:
            d.wait()
        return 0

    lax.fori_loop(0, NB, blk, 0)


def _f32(*shape):
    return jax.ShapeDtypeStruct(shape, jnp.float32)


def kernel(t, f0, index, v0, epsilon_v, epsilon_r):
    f0c = [f0[:, i] for i in range(3)]
    v0c = [v0[:, i] for i in range(3)]
    evc = [epsilon_v[:, i] for i in range(3)]
    erc = [epsilon_r[:, i] for i in range(3)]

    sums = pl.kernel(
        _sums_body,
        out_type=(_f32(NC, 3 * S), _f32(NC, 3 * S), _f32(NC, S)),
        mesh=_mesh(),
        compiler_params=_params,
        scratch_types=[
            pltpu.VMEM((2 * RB,), jnp.int32),
            pltpu.VMEM((RB,), jnp.int32),
            pltpu.VMEM((RB,), jnp.int32),
            pltpu.VMEM((RB,), jnp.int32),
            pltpu.VMEM((2 * 3 * RB,), jnp.float32),
            pltpu.VMEM((2 * 3 * RB,), jnp.float32),
            pltpu.VMEM((RB,), jnp.float32),
            pltpu.VMEM((ZL,), jnp.float32),
            pltpu.VMEM_SHARED((3 * S,), jnp.float32),
            pltpu.VMEM_SHARED((3 * S,), jnp.float32),
            pltpu.VMEM_SHARED((S,), jnp.float32),
            pltpu.SemaphoreType.DMA,
            pltpu.SemaphoreType.DMA,
        ],
    )
    psum_v, psum_r, pcnt = sums(index, *evc, *erc)

    main = pl.kernel(
        _main_body,
        out_type=tuple([_f32(N)] * 12 + [_f32(NC, 3 * S)]),
        mesh=_mesh(),
        compiler_params=_params,
        scratch_types=[
            pltpu.VMEM((2 * RB,), jnp.int32),
            pltpu.VMEM((RB,), jnp.int32),
            pltpu.VMEM((RB,), jnp.int32),
            pltpu.VMEM((RB,), jnp.int32),
            pltpu.VMEM((2 * RB,), jnp.float32),
            pltpu.VMEM((2 * 3 * RB,), jnp.float32),
            pltpu.VMEM((2 * 3 * RB,), jnp.float32),
            pltpu.VMEM((2 * 3 * RB,), jnp.float32),
            pltpu.VMEM((3 * RB,), jnp.float32),
            pltpu.VMEM((3 * RB,), jnp.float32),
            pltpu.VMEM((3 * RB,), jnp.float32),
            pltpu.VMEM((3 * RB,), jnp.float32),
            pltpu.VMEM((3 * RB,), jnp.float32),
            pltpu.VMEM((3 * RB,), jnp.float32),
            pltpu.VMEM((RB,), jnp.float32),
            pltpu.VMEM((RB,), jnp.float32),
            pltpu.VMEM((RB,), jnp.float32),
            pltpu.VMEM((RB,), jnp.float32),
            pltpu.VMEM((ZL,), jnp.float32),
            pltpu.VMEM((CL,), jnp.float32),
            pltpu.VMEM((CL,), jnp.float32),
            pltpu.VMEM((CL,), jnp.float32),
            pltpu.VMEM_SHARED((3 * S,), jnp.float32),
            pltpu.VMEM_SHARED((3 * S,), jnp.float32),
            pltpu.VMEM_SHARED((3 * S,), jnp.float32),
        ] + [pltpu.SemaphoreType.DMA] * 5,
    )
    outs = main(t, index, *v0c, *evc, *erc, psum_v, psum_r, pcnt)
    vt = outs[0:3]
    ec = outs[3:6]
    rc = outs[6:9]
    rp = outs[9:12]
    psum_rt = outs[12]

    final = pl.kernel(
        _final_body,
        out_type=tuple([_f32(N)] * 6),
        mesh=_mesh(),
        compiler_params=_params,
        scratch_types=[
            pltpu.VMEM((2 * RB,), jnp.int32),
            pltpu.VMEM((RB,), jnp.int32),
            pltpu.VMEM((RB,), jnp.int32),
            pltpu.VMEM((RB,), jnp.int32),
            pltpu.VMEM((2 * 3 * RB,), jnp.float32),
            pltpu.VMEM((2 * 3 * RB,), jnp.float32),
            pltpu.VMEM((3 * RB,), jnp.float32),
            pltpu.VMEM((3 * RB,), jnp.float32),
            pltpu.VMEM((3 * RB,), jnp.float32),
            pltpu.VMEM((CL,), jnp.float32),
            pltpu.VMEM((CL,), jnp.float32),
            pltpu.VMEM((CL,), jnp.float32),
            pltpu.VMEM_SHARED((3 * S,), jnp.float32),
        ] + [pltpu.SemaphoreType.DMA] * 4,
    )
    fouts = final(*f0c, *rp, index, psum_rt, pcnt)
    rt = fouts[0:3]
    ft = fouts[3:6]

    stack = lambda cols: jnp.stack(cols, axis=1)
    return (stack(ft), stack(vt), stack(ec), stack(rc), stack(rt))
